# Initial kernel scaffold; baseline (speedup 1.0000x reference)
import jax
import jax.numpy as jnp
from jax.experimental import pallas as pl


def kernel(classifications, regressions, anchors):
    raise NotImplementedError("write your pallas kernel here")


# speedup vs baseline: 1.0003x; 1.0003x over previous
import functools
import jax
import jax.numpy as jnp
from jax.experimental import pallas as pl

N = 20000
C = 80
K = 300
CLS_THRES = 0.05
IOU_THRES = 0.5
IMG_H = 640.0
IMG_W = 640.0

NPAD = 20096
ROWS = 157
LANES = 128


def _decode_body(ax1, ay1, ax2, ay2, dx, dy, dw, dh, x1o, y1o, x2o, y2o):
    wa = ax2[...] - ax1[...]
    ha = ay2[...] - ay1[...]
    cxa = ax1[...] + 0.5 * wa
    cya = ay1[...] + 0.5 * ha
    cx = cxa + dx[...] * 0.1 * wa
    cy = cya + dy[...] * 0.1 * ha
    w = jnp.exp(dw[...] * 0.2) * wa
    h = jnp.exp(dh[...] * 0.2) * ha
    x1o[...] = jnp.clip(cx - 0.5 * w, 0.0, IMG_W)
    y1o[...] = jnp.clip(cy - 0.5 * h, 0.0, IMG_H)
    x2o[...] = jnp.clip(cx + 0.5 * w, 0.0, IMG_W)
    y2o[...] = jnp.clip(cy + 0.5 * h, 0.0, IMG_H)


def _decode_boxes(anchors, regressions):
    def planar(a):
        pads = jnp.zeros((NPAD - N,), a.dtype)
        return [jnp.concatenate([a[:, i], pads]).reshape(ROWS, LANES) for i in range(4)]

    ins = planar(anchors) + planar(regressions)
    outs = pl.pallas_call(
        _decode_body,
        out_shape=[jax.ShapeDtypeStruct((ROWS, LANES), jnp.float32)] * 4,
    )(*ins)
    return jnp.stack([o.reshape(NPAD)[:N] for o in outs], axis=-1)


def _pairwise_iou_(b):
    area = (b[:, 2] - b[:, 0]) * (b[:, 3] - b[:, 1])
    lt = jnp.maximum(b[:, None, :2], b[None, :, :2])
    rb = jnp.minimum(b[:, None, 2:], b[None, :, 2:])
    wh = jnp.clip(rb - lt, 0.0)
    inter = wh[..., 0] * wh[..., 1]
    return inter / (area[:, None] + area[None, :] - inter + 1e-8)


def _nms_one_class_(boxes, scores):
    masked = jnp.where(scores > CLS_THRES, scores, -1.0)
    top_s, top_i = jax.lax.top_k(masked, K)
    cand = jnp.take(boxes, top_i, axis=0)
    iou = _pairwise_iou_(cand)
    ar = jnp.arange(K)

    def body(i, state):
        keep, supp = state
        is_live = jnp.logical_and(jnp.logical_not(supp[i]), top_s[i] > 0.0)
        keep = keep.at[i].set(is_live)
        sup_row = jnp.logical_and(iou[i] > IOU_THRES, ar > i)
        supp = jnp.where(jnp.logical_and(is_live, sup_row), True, supp)
        return (keep, supp)

    keep, _ = jax.lax.fori_loop(0, K, body, (jnp.zeros((K,), bool), jnp.zeros((K,), bool)))
    out_s = jnp.where(keep, top_s, -1.0)
    return out_s, cand


@jax.jit
def kernel(classifications, regressions, anchors):
    pred_boxes = _decode_boxes(anchors[0], regressions[0])
    cls = classifications[0]
    per_cls_scores, per_cls_boxes = jax.vmap(_nms_one_class_, in_axes=(None, 1))(pred_boxes, cls)
    flat_scores = per_cls_scores.reshape(-1)
    flat_boxes = per_cls_boxes.reshape(-1, 4)
    final_scores, flat_idx = jax.lax.top_k(flat_scores, K)
    final_labels = flat_idx // K
    final_boxes = jnp.take(flat_boxes, flat_idx, axis=0)
    return (final_scores, final_labels, final_boxes)


# speedup vs baseline: 2.5506x; 2.5499x over previous
import functools
import numpy as np
import jax
import jax.numpy as jnp
from jax import lax
from jax.experimental import pallas as pl
from jax.experimental.pallas import tpu as pltpu
from jax.experimental.pallas import tpu_sc as plsc

N = 20000
C = 80
K = 300
CLS_THRES = 0.05
IMG_H = 640.0
IMG_W = 640.0

NPAD = 20096
ROWS = 157
LANES = 128

NW = 32
B = 304
BG = 384
SORTN = 512
CAP = 496
BLK = 10
NVEC = N // 16
NBLK = NVEC // BLK
NVEC_M = (C * B) // 16
NBLK_M = NVEC_M // BLK
SIGN = np.uint32(0x80000000)
IMIN = np.int32(-2**31)


def _li():
    return lax.iota(jnp.int32, 16)


def _sload(ref, i):
    return ref[pl.ds(i, 16)][0]


def _bc(x, dtype):
    return jnp.full((16,), x, dtype=dtype)


_GDN = lax.GatherDimensionNumbers(
    offset_dims=(), collapsed_slice_dims=(0,), start_index_map=(0,))


def _laneperm(x, idx):
    return lax.gather(x, idx.reshape(16, 1), _GDN, (1,),
                      mode=lax.GatherScatterMode.PROMISE_IN_BOUNDS)


def _tree_sum(x):
    li = _li()
    for d in (1, 2, 4, 8):
        x = x + _laneperm(x, li ^ d)
    return x


def _tree_max(x):
    li = _li()
    for d in (1, 2, 4, 8):
        x = jnp.maximum(x, _laneperm(x, li ^ d))
    return x


def _incl_prefix(m_i32):
    li = _li()
    x = m_i32
    for d in (1, 2, 4, 8):
        sh = _laneperm(x, jnp.maximum(li - d, 0))
        x = x + jnp.where(li >= d, sh, 0)
    return x


def _compact_src(incl):
    li = _li()
    pos = jnp.zeros((16,), jnp.int32)
    tgt = li + 1
    for b in (8, 4, 2, 1):
        g = _laneperm(incl, pos + (b - 1))
        pos = jnp.where(g < tgt, pos + b, pos)
    return pos


def _uk_to_ks(thr_u32):
    return lax.convert_element_type(thr_u32 ^ SIGN, jnp.int32)


def _decode_body(ax1, ay1, ax2, ay2, dx, dy, dw, dh, x1o, y1o, x2o, y2o):
    wa = ax2[...] - ax1[...]
    ha = ay2[...] - ay1[...]
    cxa = ax1[...] + 0.5 * wa
    cya = ay1[...] + 0.5 * ha
    cx = cxa + dx[...] * 0.1 * wa
    cy = cya + dy[...] * 0.1 * ha
    w = jnp.exp(dw[...] * 0.2) * wa
    h = jnp.exp(dh[...] * 0.2) * ha
    x1o[...] = jnp.clip(cx - 0.5 * w, 0.0, IMG_W)
    y1o[...] = jnp.clip(cy - 0.5 * h, 0.0, IMG_H)
    x2o[...] = jnp.clip(cx + 0.5 * w, 0.0, IMG_W)
    y2o[...] = jnp.clip(cy + 0.5 * h, 0.0, IMG_H)


def _decode_boxes(anchors, regressions):
    def planar(a):
        pads = jnp.zeros((NPAD - N,), a.dtype)
        return [jnp.concatenate([a[:, i], pads]).reshape(ROWS, LANES)
                for i in range(4)]

    ins = planar(anchors) + planar(regressions)
    outs = pl.pallas_call(
        _decode_body,
        out_shape=[jax.ShapeDtypeStruct((ROWS, LANES), jnp.float32)] * 4,
    )(*ins)
    return [o.reshape(NPAD) for o in outs]


def _fill_ks(src_ref, ks_ref, bmax_ref, nblk, mask_thres):
    li = _li()

    def blk(bi, _):
        base = bi * BLK
        acc = _bc(IMIN, jnp.int32)
        for jj in range(BLK):
            o = pl.ds(16 * (base + jj), 16)
            s = src_ref[o]
            if mask_thres:
                s = jnp.where(s > CLS_THRES, s, jnp.float32(-1.0))
            ks = lax.bitcast_convert_type(s, jnp.int32)
            ks_ref[o] = ks
            acc = jnp.maximum(acc, ks)
        bm = _tree_max(acc)
        wd = bmax_ref[pl.ds(bi, 16)]
        bmax_ref[pl.ds(bi, 16)] = jnp.where(li == 0, bm, wd)
        return 0
    lax.fori_loop(0, nblk, blk, 0)


def _count_ge(ks_ref, bmax_ref, nblk, thr_ks, cntbuf):
    cntbuf[pl.ds(0, 16)] = jnp.zeros((16,), jnp.int32)
    thrv = _bc(thr_ks, jnp.int32)

    def blk(bi, _):
        bm = _sload(bmax_ref, bi)

        @pl.when(bm >= thr_ks)
        def _():
            base = bi * BLK
            acc = jnp.zeros((16,), jnp.int32)
            for jj in range(BLK):
                u = ks_ref[pl.ds(16 * (base + jj), 16)]
                acc = acc + jnp.where(u >= thrv, 1, 0)
            cntbuf[pl.ds(0, 16)] = cntbuf[pl.ds(0, 16)] + acc
        return 0
    lax.fori_loop(0, nblk, blk, 0)


def _append(dstK, dstI, offbuf, ks, idxv, mask):
    li = _li()
    incl = _incl_prefix(jnp.where(mask, 1, 0))
    cnt = lax.rev(incl, (0,))[0]
    src = _compact_src(incl)
    ck = _laneperm(ks, src)
    ci = _laneperm(idxv, src)

    @pl.when(cnt > 0)
    def _():
        cntv = _bc(cnt, jnp.int32)
        offv = offbuf[pl.ds(0, 16)]
        off = jnp.minimum(offv[0], SORTN - 16)
        sel = li < cntv
        dstK[pl.ds(off, 16)] = jnp.where(sel, ck, dstK[pl.ds(off, 16)])
        dstI[pl.ds(off, 16)] = jnp.where(sel, ci, dstI[pl.ds(off, 16)])
        offbuf[pl.ds(0, 16)] = offv + cntv


def _bisect_select(ks_ref, bmax_ref, nblk, sortK_ref, sortI_ref, cntbuf,
                   gob, eob):
    li = _li()
    zk = _bc(IMIN, jnp.int32)
    zi = jnp.zeros((16,), jnp.int32)

    def zbody(v, _):
        sortK_ref[pl.ds(16 * v, 16)] = zk
        sortI_ref[pl.ds(16 * v, 16)] = zi
        return 0
    lax.fori_loop(0, SORTN // 16, zbody, 0)

    lo = jnp.uint32(0)
    hi = jnp.uint32(0xFFFFFFFF)
    done = jnp.bool_(False)
    GTH = jnp.uint32(0)
    EQLO = jnp.uint32(0)
    capped = jnp.bool_(False)

    for _step in range(33):
        active = jnp.logical_not(done)
        mid = lo + np.uint32(1) + lax.shift_right_logical(
            hi - lo - np.uint32(1), np.uint32(1))
        mid_ks = _uk_to_ks(mid)

        @pl.when(active)
        def _(mid_ks=mid_ks):
            _count_ge(ks_ref, bmax_ref, nblk, mid_ks, cntbuf)

        cnt_mid = _tree_sum(cntbuf[pl.ds(0, 16)])[0]
        geK = cnt_mid >= K
        hitw = jnp.logical_and(active,
                               jnp.logical_and(geK, cnt_mid <= CAP))
        lo2 = jnp.where(jnp.logical_and(active, geK), mid, lo)
        hi2 = jnp.where(jnp.logical_and(active, jnp.logical_not(geK)),
                        mid - np.uint32(1), hi)
        conv = jnp.logical_and(jnp.logical_and(active,
                                               jnp.logical_not(hitw)),
                               lo2 == hi2)
        GTH = jnp.where(hitw, mid - np.uint32(1),
                        jnp.where(conv, lo2, GTH))
        EQLO = jnp.where(hitw, mid, jnp.where(conv, lo2, EQLO))
        capped = jnp.logical_or(capped, conv)
        done = jnp.logical_or(done, jnp.logical_or(hitw, conv))
        lo, hi = lo2, hi2

    m1_ks = _uk_to_ks(GTH + np.uint32(1))

    @pl.when(capped)
    def _():
        _count_ge(ks_ref, bmax_ref, nblk, m1_ks, cntbuf)

    m = jnp.where(capped, _tree_sum(cntbuf[pl.ds(0, 16)])[0], 0)

    gob[pl.ds(0, 16)] = jnp.zeros((16,), jnp.int32)
    eob[pl.ds(0, 16)] = _bc(m, jnp.int32)

    gth_ks = _uk_to_ks(GTH)
    eqlo_ks = _uk_to_ks(EQLO)
    gthv = _bc(gth_ks, jnp.int32)
    eqlov = _bc(eqlo_ks, jnp.int32)

    def cblk(bi, _):
        bm = _sload(bmax_ref, bi)

        @pl.when(bm >= eqlo_ks)
        def _():
            base = bi * BLK
            for jj in range(BLK):
                u = ks_ref[pl.ds(16 * (base + jj), 16)]
                idxv = li + 16 * (base + jj)
                gt = u > gthv
                _append(sortK_ref, sortI_ref, gob, u, idxv, gt)
                eq = jnp.logical_and(u >= eqlov, u <= gthv)
                ape = _sload(eob, 0) < K
                apev = _bc(jnp.where(ape, 1, 0), jnp.int32)
                eqm = (jnp.where(eq, 1, 0) & apev) == 1
                _append(sortK_ref, sortI_ref, eob, u, idxv, eqm)
        return 0
    lax.fori_loop(0, nblk, cblk, 0)


def _bitonic_sort_512(sortK_ref, sortI_ref):
    li = _li()

    def xlayer(kk, j, logkk, logj):
        jr = j // 16
        logjr = logj - 4

        def body(i, _):
            r1 = (i & (jr - 1)) | ((i >> logjr) << (logjr + 1))
            dscv = _bc((r1 >> (logkk - 4)) & 1, jnp.int32)
            o1 = 16 * r1
            o2 = o1 + j
            K1 = sortK_ref[pl.ds(o1, 16)]
            K2 = sortK_ref[pl.ds(o2, 16)]
            I1 = sortI_ref[pl.ds(o1, 16)]
            I2 = sortI_ref[pl.ds(o2, 16)]
            better1 = jnp.logical_or(K1 > K2,
                                     jnp.logical_and(K1 == K2, I1 < I2))
            keep = (jnp.where(better1, 1, 0) ^ dscv) == 1
            sortK_ref[pl.ds(o1, 16)] = jnp.where(keep, K1, K2)
            sortK_ref[pl.ds(o2, 16)] = jnp.where(keep, K2, K1)
            sortI_ref[pl.ds(o1, 16)] = jnp.where(keep, I1, I2)
            sortI_ref[pl.ds(o2, 16)] = jnp.where(keep, I2, I1)
            return 0
        lax.fori_loop(0, 16, body, 0)

    def llayer(kk, jl, logkk):
        pidx = li ^ jl
        upper = (li & jl) != 0

        upv = jnp.where(upper, 1, 0)

        def body(r, _):
            o = 16 * r
            Kv = sortK_ref[pl.ds(o, 16)]
            Iv = sortI_ref[pl.ds(o, 16)]
            pK = _laneperm(Kv, pidx)
            pI = _laneperm(Iv, pidx)
            if logkk < 4:
                flipv = ((li >> logkk) & 1) ^ upv
            else:
                flipv = _bc((r >> (logkk - 4)) & 1, jnp.int32) ^ upv
            better = jnp.logical_or(Kv > pK,
                                    jnp.logical_and(Kv == pK, Iv < pI))
            keep = (jnp.where(better, 1, 0) ^ flipv) == 1
            sortK_ref[pl.ds(o, 16)] = jnp.where(keep, Kv, pK)
            sortI_ref[pl.ds(o, 16)] = jnp.where(keep, Iv, pI)
            return 0
        lax.fori_loop(0, 32, body, 0)

    kk = 2
    while kk <= SORTN:
        logkk = kk.bit_length() - 1
        j = kk // 2
        while j >= 1:
            if j >= 16:
                xlayer(kk, j, logkk, j.bit_length() - 1)
            else:
                llayer(kk, j, logkk)
            j //= 2
        kk *= 2


def _extract_sorted(sortK_ref, sortI_ref, candI2_ref, tops_ref):
    for o in range(BG // 16):
        ii = sortI_ref[pl.ds(16 * o, 16)]
        candI2_ref[o // 8, pl.ds((o % 8) * 16, 16)] = ii
        if o < B // 16:
            ki = sortK_ref[pl.ds(16 * o, 16)]
            tops_ref[pl.ds(16 * o, 16)] = lax.bitcast_convert_type(
                ki, jnp.float32)


def _gather_boxes(planes, candI2_ref, dsts, sem):
    cps = []
    for g in range(3):
        for plane, dst in zip(planes, dsts):
            cp = pltpu.make_async_copy(plane.at[candI2_ref.at[g]],
                                       dst.at[pl.ds(128 * g, 128)], sem)
            cp.start()
            cps.append(cp)
    for cp in cps:
        cp.wait()


def _sc_main_body(scoresT, x1p, y1p, x2p, y2p,
                  outS, ob1, ob2, ob3, ob4,
                  scores_v, ksbuf, bmax, sortK, sortI, candI2, tops,
                  bx1, by1, bx2, by2, ar, obits, suppw, keepw, outbuf,
                  cntbuf, gob, eob, sem):
    wid = lax.axis_index("s") * 2 + lax.axis_index("c")
    li = _li()
    one_shift = lax.shift_left(jnp.ones((16,), jnp.int32), li)
    zi = jnp.zeros((16,), jnp.int32)

    def class_body(t, _):
        c = wid + NW * t

        @pl.when(c < C)
        def _():
            pltpu.sync_copy(scoresT.at[pl.ds(c * N, N)], scores_v)
            _fill_ks(scores_v, ksbuf, bmax, NBLK, True)
            _bisect_select(ksbuf, bmax, NBLK, sortK, sortI, cntbuf, gob, eob)
            _bitonic_sort_512(sortK, sortI)
            _extract_sorted(sortK, sortI, candI2, tops)
            _gather_boxes((x1p, y1p, x2p, y2p), candI2,
                          (bx1, by1, bx2, by2), sem)

            def area_body(v, _):
                o = pl.ds(16 * v, 16)
                ar[o] = (bx2[o] - bx1[o]) * (by2[o] - by1[o])
                return 0
            lax.fori_loop(0, B // 16, area_body, 0)

            def row_body(i, _):
                w0 = i >> 4
                xi1 = _sload(bx1, i)
                yi1 = _sload(by1, i)
                xi2 = _sload(bx2, i)
                yi2 = _sload(by2, i)
                ai = _sload(ar, i)

                def col_body(v, _):
                    o = pl.ds(16 * v, 16)
                    X1 = bx1[o]
                    Y1 = by1[o]
                    X2 = bx2[o]
                    Y2 = by2[o]
                    A = ar[o]
                    iw = jnp.maximum(jnp.minimum(xi2, X2) - jnp.maximum(xi1, X1),
                                     0.0)
                    ih = jnp.maximum(jnp.minimum(yi2, Y2) - jnp.maximum(yi1, Y1),
                                     0.0)
                    inter = iw * ih
                    iou = inter / (ai + A - inter + 1e-8)
                    cm = jnp.logical_and(iou > 0.5, (li + 16 * v) > _bc(i, jnp.int32))
                    bits = _tree_sum(jnp.where(cm, one_shift, 0))
                    pos = i * (B // 16) + v
                    wd = obits[pl.ds(pos, 16)]
                    obits[pl.ds(pos, 16)] = jnp.where(li == 0, bits, wd)
                    return 0
                lax.fori_loop(w0, B // 16, col_body, 0)
                return 0
            lax.fori_loop(0, K, row_body, 0)

            suppw[pl.ds(0, 16)] = zi
            suppw[pl.ds(16, 16)] = zi
            suppw[pl.ds(32, 16)] = zi
            keepw[pl.ds(0, 16)] = zi
            keepw[pl.ds(16, 16)] = zi
            keepw[pl.ds(32, 16)] = zi

            def nms_body(i, _):
                w = i >> 4
                bpos = i & 15
                supv = suppw[pl.ds(w, 16)]
                sup = (supv[0] >> bpos) & 1
                live = jnp.logical_and(sup == 0, _sload(tops, i) > 0.0)
                kv = keepw[pl.ds(w, 16)]
                nkw = kv[0] | (jnp.where(live, 1, 0) << bpos)
                keepw[pl.ds(w, 16)] = jnp.where(li == 0, _bc(nkw, jnp.int32),
                                                kv)

                @pl.when(live)
                def _():
                    ob0 = obits[pl.ds(i * (B // 16) + w, 16)]
                    suppw[pl.ds(w, 16)] = suppw[pl.ds(w, 16)] | ob0
                    ob1v = obits[pl.ds(i * (B // 16) + w + 16, 16)]
                    suppw[pl.ds(w + 16, 16)] = suppw[pl.ds(w + 16, 16)] | ob1v
                return 0
            lax.fori_loop(0, K, nms_body, 0)

            for v in range(B // 16):
                kvec = (_bc(_sload(keepw, v), jnp.int32) >> li) & 1
                sv = tops[pl.ds(16 * v, 16)]
                colid = li + 16 * v
                outv = jnp.where(colid < K,
                                 jnp.where(kvec == 1, sv, jnp.float32(-1.0)),
                                 jnp.float32(-0.0))
                outbuf[pl.ds(16 * v, 16)] = outv

            pltpu.sync_copy(outbuf, outS.at[pl.ds(c * B, B)])
            pltpu.sync_copy(bx1.at[pl.ds(0, B)], ob1.at[pl.ds(c * B, B)])
            pltpu.sync_copy(by1.at[pl.ds(0, B)], ob2.at[pl.ds(c * B, B)])
            pltpu.sync_copy(bx2.at[pl.ds(0, B)], ob3.at[pl.ds(c * B, B)])
            pltpu.sync_copy(by2.at[pl.ds(0, B)], ob4.at[pl.ds(c * B, B)])
        return 0

    lax.fori_loop(0, 3, class_body, 0)


_sc_main = functools.partial(
    pl.kernel,
    out_type=[jax.ShapeDtypeStruct((C * B,), jnp.float32)] * 5,
    mesh=plsc.VectorSubcoreMesh(core_axis_name="c", subcore_axis_name="s"),
    scratch_types=[
        pltpu.VMEM((N,), jnp.float32),
        pltpu.VMEM((N,), jnp.int32),
        pltpu.VMEM((NBLK + 16,), jnp.int32),
        pltpu.VMEM((SORTN,), jnp.int32),
        pltpu.VMEM((SORTN,), jnp.int32),
        pltpu.VMEM((3, 128), jnp.int32),
        pltpu.VMEM((320,), jnp.float32),
        pltpu.VMEM((BG,), jnp.float32),
        pltpu.VMEM((BG,), jnp.float32),
        pltpu.VMEM((BG,), jnp.float32),
        pltpu.VMEM((BG,), jnp.float32),
        pltpu.VMEM((BG,), jnp.float32),
        pltpu.VMEM((K * (B // 16) + 64,), jnp.int32),
        pltpu.VMEM((64,), jnp.int32),
        pltpu.VMEM((64,), jnp.int32),
        pltpu.VMEM((B,), jnp.float32),
        pltpu.VMEM((16,), jnp.int32),
        pltpu.VMEM((16,), jnp.int32),
        pltpu.VMEM((16,), jnp.int32),
        pltpu.SemaphoreType.DMA,
    ],
)(_sc_main_body)


def _sc_merge_body(flatS, fb1, fb2, fb3, fb4,
                   fscore, fidx, fo1, fo2, fo3, fo4,
                   scores_v, ksbuf, bmax, sortK, sortI, candI2, tops,
                   bx1, by1, bx2, by2, cntbuf, gob, eob, sem):
    wid = lax.axis_index("s") * 2 + lax.axis_index("c")

    @pl.when(wid == 0)
    def _():
        pltpu.sync_copy(flatS, scores_v)
        _fill_ks(scores_v, ksbuf, bmax, NBLK_M, False)
        _bisect_select(ksbuf, bmax, NBLK_M, sortK, sortI, cntbuf, gob, eob)
        _bitonic_sort_512(sortK, sortI)
        _extract_sorted(sortK, sortI, candI2, tops)
        _gather_boxes((fb1, fb2, fb3, fb4), candI2,
                      (bx1, by1, bx2, by2), sem)
        pltpu.sync_copy(tops.at[pl.ds(0, B)], fscore)
        pltpu.sync_copy(candI2.at[0], fidx.at[pl.ds(0, 128)])
        pltpu.sync_copy(candI2.at[1], fidx.at[pl.ds(128, 128)])
        pltpu.sync_copy(candI2.at[2], fidx.at[pl.ds(256, 128)])
        pltpu.sync_copy(bx1.at[pl.ds(0, B)], fo1)
        pltpu.sync_copy(by1.at[pl.ds(0, B)], fo2)
        pltpu.sync_copy(bx2.at[pl.ds(0, B)], fo3)
        pltpu.sync_copy(by2.at[pl.ds(0, B)], fo4)


_sc_merge = functools.partial(
    pl.kernel,
    out_type=[jax.ShapeDtypeStruct((B,), jnp.float32),
              jax.ShapeDtypeStruct((BG,), jnp.int32)] +
             [jax.ShapeDtypeStruct((B,), jnp.float32)] * 4,
    mesh=plsc.VectorSubcoreMesh(core_axis_name="c", subcore_axis_name="s"),
    scratch_types=[
        pltpu.VMEM((C * B,), jnp.float32),
        pltpu.VMEM((C * B,), jnp.int32),
        pltpu.VMEM((NBLK_M + 16,), jnp.int32),
        pltpu.VMEM((SORTN,), jnp.int32),
        pltpu.VMEM((SORTN,), jnp.int32),
        pltpu.VMEM((3, 128), jnp.int32),
        pltpu.VMEM((320,), jnp.float32),
        pltpu.VMEM((BG,), jnp.float32),
        pltpu.VMEM((BG,), jnp.float32),
        pltpu.VMEM((BG,), jnp.float32),
        pltpu.VMEM((BG,), jnp.float32),
        pltpu.VMEM((16,), jnp.int32),
        pltpu.VMEM((16,), jnp.int32),
        pltpu.VMEM((16,), jnp.int32),
        pltpu.SemaphoreType.DMA,
    ],
)(_sc_merge_body)


@jax.jit
def kernel(classifications, regressions, anchors):
    x1p, y1p, x2p, y2p = _decode_boxes(anchors[0], regressions[0])
    scoresT = jnp.transpose(classifications[0]).reshape(-1)
    outS, ob1, ob2, ob3, ob4 = _sc_main(scoresT, x1p, y1p, x2p, y2p)
    fs, fidx, f1, f2, f3, f4 = _sc_merge(outS, ob1, ob2, ob3, ob4)
    final_scores = fs[:K]
    final_labels = (fidx[:K] // B).astype(jnp.int32)
    final_boxes = jnp.stack([f1[:K], f2[:K], f3[:K], f4[:K]], axis=-1)
    return (final_scores, final_labels, final_boxes)


# speedup vs baseline: 3.6227x; 1.4203x over previous
import functools
import numpy as np
import jax
import jax.numpy as jnp
from jax import lax
from jax.experimental import pallas as pl
from jax.experimental.pallas import tpu as pltpu
from jax.experimental.pallas import tpu_sc as plsc

N = 20000
C = 80
K = 300
CLS_THRES = 0.05
IMG_H = 640.0
IMG_W = 640.0

NPAD = 20096
ROWS = 157
LANES = 128

NW = 32
B = 304
BG = 384
SORTN = 512
CAP = 496
FCAP = 4064
FBUF = 4112
BLK = 10
NVEC = N // 16
NBLK = NVEC // BLK
NVEC_M = (C * B) // 16
NBLK_M = NVEC_M // BLK
SIGN = np.uint32(0x80000000)
IMIN = np.int32(-2**31)


def _li():
    return lax.iota(jnp.int32, 16)


def _sload(ref, i):
    return ref[pl.ds(i, 16)][0]


def _bc(x, dtype):
    return jnp.full((16,), x, dtype=dtype)


_GDN = lax.GatherDimensionNumbers(
    offset_dims=(), collapsed_slice_dims=(0,), start_index_map=(0,))


def _laneperm(x, idx):
    return lax.gather(x, idx.reshape(16, 1), _GDN, (1,),
                      mode=lax.GatherScatterMode.PROMISE_IN_BOUNDS)


def _tree_sum(x):
    li = _li()
    for d in (1, 2, 4, 8):
        x = x + _laneperm(x, li ^ d)
    return x


def _tree_max(x):
    li = _li()
    for d in (1, 2, 4, 8):
        x = jnp.maximum(x, _laneperm(x, li ^ d))
    return x


def _tree_min(x):
    li = _li()
    for d in (1, 2, 4, 8):
        x = jnp.minimum(x, _laneperm(x, li ^ d))
    return x


def _incl_prefix(m_i32):
    li = _li()
    x = m_i32
    for d in (1, 2, 4, 8):
        sh = _laneperm(x, jnp.maximum(li - d, 0))
        x = x + jnp.where(li >= d, sh, 0)
    return x


def _compact_src(incl):
    li = _li()
    pos = jnp.zeros((16,), jnp.int32)
    tgt = li + 1
    for b in (8, 4, 2, 1):
        g = _laneperm(incl, pos + (b - 1))
        pos = jnp.where(g < tgt, pos + b, pos)
    return pos


def _uk_to_ks(thr_u32):
    return lax.convert_element_type(thr_u32 ^ SIGN, jnp.int32)


def _decode_body(ax1, ay1, ax2, ay2, dx, dy, dw, dh, x1o, y1o, x2o, y2o):
    wa = ax2[...] - ax1[...]
    ha = ay2[...] - ay1[...]
    cxa = ax1[...] + 0.5 * wa
    cya = ay1[...] + 0.5 * ha
    cx = cxa + dx[...] * 0.1 * wa
    cy = cya + dy[...] * 0.1 * ha
    w = jnp.exp(dw[...] * 0.2) * wa
    h = jnp.exp(dh[...] * 0.2) * ha
    x1o[...] = jnp.clip(cx - 0.5 * w, 0.0, IMG_W)
    y1o[...] = jnp.clip(cy - 0.5 * h, 0.0, IMG_H)
    x2o[...] = jnp.clip(cx + 0.5 * w, 0.0, IMG_W)
    y2o[...] = jnp.clip(cy + 0.5 * h, 0.0, IMG_H)


def _decode_boxes(anchors, regressions):
    def planar(a):
        pads = jnp.zeros((NPAD - N,), a.dtype)
        return [jnp.concatenate([a[:, i], pads]).reshape(ROWS, LANES)
                for i in range(4)]

    ins = planar(anchors) + planar(regressions)
    outs = pl.pallas_call(
        _decode_body,
        out_shape=[jax.ShapeDtypeStruct((ROWS, LANES), jnp.float32)] * 4,
    )(*ins)
    return [o.reshape(NPAD) for o in outs]


def _fill_ks(src_ref, ks_ref, bmax_ref, nblk, mask_thres):
    li = _li()

    def blk(bi, _):
        base = bi * BLK
        acc = _bc(IMIN, jnp.int32)
        for jj in range(BLK):
            o = pl.ds(16 * (base + jj), 16)
            s = src_ref[o]
            if mask_thres:
                s = jnp.where(s > CLS_THRES, s, jnp.float32(-1.0))
            ks = lax.bitcast_convert_type(s, jnp.int32)
            ks_ref[o] = ks
            acc = jnp.maximum(acc, ks)
        bm = _tree_max(acc)
        wd = bmax_ref[pl.ds(bi, 16)]
        bmax_ref[pl.ds(bi, 16)] = jnp.where(li == 0, bm, wd)
        return 0
    lax.fori_loop(0, nblk, blk, 0)


def _count_ge(ks_ref, bmax_ref, nblk, thr_ks, cntbuf):
    cntbuf[pl.ds(0, 16)] = jnp.zeros((16,), jnp.int32)
    thrv = _bc(thr_ks, jnp.int32)

    def blk(bi, _):
        bm = _sload(bmax_ref, bi)

        @pl.when(bm >= thr_ks)
        def _():
            base = bi * BLK
            acc = jnp.zeros((16,), jnp.int32)
            for jj in range(BLK):
                u = ks_ref[pl.ds(16 * (base + jj), 16)]
                acc = acc + jnp.where(u >= thrv, 1, 0)
            cntbuf[pl.ds(0, 16)] = cntbuf[pl.ds(0, 16)] + acc
        return 0
    lax.fori_loop(0, nblk, blk, 0)


def _append(dstK, dstI, offbuf, ks, idxv, mask, cap=SORTN):
    li = _li()
    incl = _incl_prefix(jnp.where(mask, 1, 0))
    cnt = lax.rev(incl, (0,))[0]
    src = _compact_src(incl)
    ck = _laneperm(ks, src)
    ci = _laneperm(idxv, src)

    @pl.when(cnt > 0)
    def _():
        cntv = _bc(cnt, jnp.int32)
        offv = offbuf[pl.ds(0, 16)]
        off = jnp.minimum(offv[0], cap - 16)
        sel = li < cntv
        dstK[pl.ds(off, 16)] = jnp.where(sel, ck, dstK[pl.ds(off, 16)])
        dstI[pl.ds(off, 16)] = jnp.where(sel, ci, dstI[pl.ds(off, 16)])
        offbuf[pl.ds(0, 16)] = offv + cntv


def _ks_to_u(ks):
    return lax.convert_element_type(ks, jnp.uint32) ^ SIGN


def _bisect_core(count_fn, cntbuf, lo0, hi0):
    def step(_s, c):
        lo, hi, done_i, GTH, EQLO, capped_i = c
        active = done_i == 0
        mid = lo + np.uint32(1) + lax.shift_right_logical(
            hi - lo - np.uint32(1), np.uint32(1))
        mid_ks = _uk_to_ks(mid)

        @pl.when(active)
        def _():
            count_fn(mid_ks)

        cnt = _tree_sum(cntbuf[pl.ds(0, 16)])[0]
        geK = cnt >= K
        hitw = jnp.logical_and(active, jnp.logical_and(geK, cnt <= CAP))
        lo2 = jnp.where(jnp.logical_and(active, geK), mid, lo)
        hi2 = jnp.where(jnp.logical_and(active, jnp.logical_not(geK)),
                        mid - np.uint32(1), hi)
        conv = jnp.logical_and(jnp.logical_and(active,
                                               jnp.logical_not(hitw)),
                               lo2 == hi2)
        GTH2 = jnp.where(hitw, mid - np.uint32(1),
                         jnp.where(conv, lo2, GTH))
        EQLO2 = jnp.where(hitw, mid, jnp.where(conv, lo2, EQLO))
        capped2 = jnp.where(conv, 1, capped_i)
        done2 = jnp.where(jnp.logical_or(hitw, conv), 1, done_i)
        return (lo2, hi2, done2, GTH2, EQLO2, capped2)

    _, _, _, GTH, EQLO, capped_i = lax.fori_loop(
        0, 33, step, (lo0, hi0, jnp.int32(0), lo0, lo0, jnp.int32(0)))

    m1_ks = _uk_to_ks(GTH + np.uint32(1))

    @pl.when(capped_i == 1)
    def _():
        count_fn(m1_ks)

    m = jnp.where(capped_i == 1, _tree_sum(cntbuf[pl.ds(0, 16)])[0], 0)
    return GTH, EQLO, capped_i, m


def _compact_pair(dstK, dstI, gob, eob, u, idxv, gthv, eqlov):
    _append(dstK, dstI, gob, u, idxv, u > gthv)
    eq = jnp.logical_and(u >= eqlov, u <= gthv)
    ape = _sload(eob, 0) < K
    apev = _bc(jnp.where(ape, 1, 0), jnp.int32)
    eqm = (jnp.where(eq, 1, 0) & apev) == 1
    _append(dstK, dstI, eob, u, idxv, eqm)


def _bisect_select(ks_ref, bmax_ref, nblk, nvec, sortK_ref, sortI_ref,
                   cntbuf, gob, eob, fbK, fbI, fob):
    li = _li()
    zk = _bc(IMIN, jnp.int32)
    zi = jnp.zeros((16,), jnp.int32)

    def zbody(v, _):
        sortK_ref[pl.ds(16 * v, 16)] = zk
        sortI_ref[pl.ds(16 * v, 16)] = zi
        return 0
    lax.fori_loop(0, SORTN // 16, zbody, 0)

    accmin = _bc(np.int32(2**31 - 1), jnp.int32)
    accmax = _bc(IMIN, jnp.int32)
    for v in range((nblk + 15) // 16):
        w = bmax_ref[pl.ds(16 * v, 16)]
        valid = (li + 16 * v) < nblk
        accmin = jnp.minimum(accmin, jnp.where(valid, w, np.int32(2**31 - 1)))
        accmax = jnp.maximum(accmax, jnp.where(valid, w, IMIN))
    t0k = lax.rev(_tree_min(accmin), (0,))[0]
    gmaxu = _ks_to_u(lax.rev(_tree_max(accmax), (0,))[0])

    _count_ge(ks_ref, bmax_ref, nblk, t0k, cntbuf)
    cnt0 = _tree_sum(cntbuf[pl.ds(0, 16)])[0]
    fast = jnp.logical_and(cnt0 >= K, cnt0 <= FCAP)

    @pl.when(fast)
    def _():
        fob[pl.ds(0, 16)] = zi
        t0v = _bc(t0k, jnp.int32)

        def fblk(j, _):
            u = ks_ref[pl.ds(16 * j, 16)]
            _append(fbK, fbI, fob, u, li + 16 * j, u >= t0v, cap=FBUF)
            return 0
        lax.fori_loop(0, nvec, fblk, 0)
        flen = _sload(fob, 0)
        fbK[pl.ds(flen, 16)] = zk
        fbI[pl.ds(flen, 16)] = zi
        nvb = (flen + 15) >> 4

        def bcount(thr_ks):
            thrv = _bc(thr_ks, jnp.int32)

            def b(j, a):
                return a + jnp.where(fbK[pl.ds(16 * j, 16)] >= thrv, 1, 0)
            cntbuf[pl.ds(0, 16)] = lax.fori_loop(
                0, nvb, b, jnp.zeros((16,), jnp.int32))

        GTH, EQLO, capped_i, m = _bisect_core(bcount, cntbuf,
                                              _ks_to_u(t0k), gmaxu)
        gob[pl.ds(0, 16)] = zi
        eob[pl.ds(0, 16)] = _bc(m, jnp.int32)
        gthv = _bc(_uk_to_ks(GTH), jnp.int32)
        eqlov = _bc(_uk_to_ks(EQLO), jnp.int32)

        def cb(j, _):
            u = fbK[pl.ds(16 * j, 16)]
            iv = fbI[pl.ds(16 * j, 16)]
            _compact_pair(sortK_ref, sortI_ref, gob, eob, u, iv, gthv, eqlov)
            return 0
        lax.fori_loop(0, nvb, cb, 0)

    @pl.when(jnp.logical_not(fast))
    def _():
        def fcount(thr_ks):
            _count_ge(ks_ref, bmax_ref, nblk, thr_ks, cntbuf)

        GTH, EQLO, capped_i, m = _bisect_core(
            fcount, cntbuf, np.uint32(0x3F800000), gmaxu)
        gob[pl.ds(0, 16)] = zi
        eob[pl.ds(0, 16)] = _bc(m, jnp.int32)
        eqlo_ks = _uk_to_ks(EQLO)
        gthv = _bc(_uk_to_ks(GTH), jnp.int32)
        eqlov = _bc(eqlo_ks, jnp.int32)

        def cblk(bi, _):
            bm = _sload(bmax_ref, bi)

            @pl.when(bm >= eqlo_ks)
            def _():
                base = bi * BLK
                for jj in range(BLK):
                    u = ks_ref[pl.ds(16 * (base + jj), 16)]
                    idxv = li + 16 * (base + jj)
                    _compact_pair(sortK_ref, sortI_ref, gob, eob, u, idxv,
                                  gthv, eqlov)
            return 0
        lax.fori_loop(0, nblk, cblk, 0)


def _bitonic_sort_512(sortK_ref, sortI_ref):
    li = _li()

    def xlayer(kk, j, logkk, logj):
        jr = j // 16
        logjr = logj - 4

        def body(i, _):
            r1 = (i & (jr - 1)) | ((i >> logjr) << (logjr + 1))
            dscv = _bc((r1 >> (logkk - 4)) & 1, jnp.int32)
            o1 = 16 * r1
            o2 = o1 + j
            K1 = sortK_ref[pl.ds(o1, 16)]
            K2 = sortK_ref[pl.ds(o2, 16)]
            I1 = sortI_ref[pl.ds(o1, 16)]
            I2 = sortI_ref[pl.ds(o2, 16)]
            better1 = jnp.logical_or(K1 > K2,
                                     jnp.logical_and(K1 == K2, I1 < I2))
            keep = (jnp.where(better1, 1, 0) ^ dscv) == 1
            sortK_ref[pl.ds(o1, 16)] = jnp.where(keep, K1, K2)
            sortK_ref[pl.ds(o2, 16)] = jnp.where(keep, K2, K1)
            sortI_ref[pl.ds(o1, 16)] = jnp.where(keep, I1, I2)
            sortI_ref[pl.ds(o2, 16)] = jnp.where(keep, I2, I1)
            return 0
        lax.fori_loop(0, 16, body, 0)

    def llayer(kk, jl, logkk):
        pidx = li ^ jl
        upper = (li & jl) != 0

        upv = jnp.where(upper, 1, 0)

        def body(r, _):
            o = 16 * r
            Kv = sortK_ref[pl.ds(o, 16)]
            Iv = sortI_ref[pl.ds(o, 16)]
            pK = _laneperm(Kv, pidx)
            pI = _laneperm(Iv, pidx)
            if logkk < 4:
                flipv = ((li >> logkk) & 1) ^ upv
            else:
                flipv = _bc((r >> (logkk - 4)) & 1, jnp.int32) ^ upv
            better = jnp.logical_or(Kv > pK,
                                    jnp.logical_and(Kv == pK, Iv < pI))
            keep = (jnp.where(better, 1, 0) ^ flipv) == 1
            sortK_ref[pl.ds(o, 16)] = jnp.where(keep, Kv, pK)
            sortI_ref[pl.ds(o, 16)] = jnp.where(keep, Iv, pI)
            return 0
        lax.fori_loop(0, 32, body, 0)

    kk = 2
    while kk <= SORTN:
        logkk = kk.bit_length() - 1
        j = kk // 2
        while j >= 1:
            if j >= 16:
                xlayer(kk, j, logkk, j.bit_length() - 1)
            else:
                llayer(kk, j, logkk)
            j //= 2
        kk *= 2


def _extract_sorted(sortK_ref, sortI_ref, candI2_ref, tops_ref):
    for o in range(BG // 16):
        ii = sortI_ref[pl.ds(16 * o, 16)]
        candI2_ref[o // 8, pl.ds((o % 8) * 16, 16)] = ii
        if o < B // 16:
            ki = sortK_ref[pl.ds(16 * o, 16)]
            tops_ref[pl.ds(16 * o, 16)] = lax.bitcast_convert_type(
                ki, jnp.float32)


def _gather_boxes(planes, candI2_ref, dsts, sem):
    cps = []
    for g in range(3):
        for plane, dst in zip(planes, dsts):
            cp = pltpu.make_async_copy(plane.at[candI2_ref.at[g]],
                                       dst.at[pl.ds(128 * g, 128)], sem)
            cp.start()
            cps.append(cp)
    for cp in cps:
        cp.wait()


def _sc_main_body(scoresT, x1p, y1p, x2p, y2p,
                  outS, ob1, ob2, ob3, ob4,
                  scores_v, ksbuf, bmax, sortK, sortI, candI2, tops,
                  bx1, by1, bx2, by2, ar, obits, suppw, keepw, outbuf,
                  cntbuf, gob, eob, fbK, fbI, fob, sem):
    wid = lax.axis_index("s") * 2 + lax.axis_index("c")
    li = _li()
    one_shift = lax.shift_left(jnp.ones((16,), jnp.int32), li)
    zi = jnp.zeros((16,), jnp.int32)

    def class_body(t, _):
        c = wid + NW * t

        @pl.when(c < C)
        def _():
            pltpu.sync_copy(scoresT.at[pl.ds(c * N, N)], scores_v)
            _fill_ks(scores_v, ksbuf, bmax, NBLK, True)
            _bisect_select(ksbuf, bmax, NBLK, NVEC, sortK, sortI, cntbuf,
                           gob, eob, fbK, fbI, fob)
            _bitonic_sort_512(sortK, sortI)
            _extract_sorted(sortK, sortI, candI2, tops)
            _gather_boxes((x1p, y1p, x2p, y2p), candI2,
                          (bx1, by1, bx2, by2), sem)

            def area_body(v, _):
                o = pl.ds(16 * v, 16)
                ar[o] = (bx2[o] - bx1[o]) * (by2[o] - by1[o])
                return 0
            lax.fori_loop(0, B // 16, area_body, 0)

            def row_body(i, _):
                w0 = i >> 4
                xi1 = _sload(bx1, i)
                yi1 = _sload(by1, i)
                xi2 = _sload(bx2, i)
                yi2 = _sload(by2, i)
                ai = _sload(ar, i)

                def col_body(v, _):
                    o = pl.ds(16 * v, 16)
                    X1 = bx1[o]
                    Y1 = by1[o]
                    X2 = bx2[o]
                    Y2 = by2[o]
                    A = ar[o]
                    iw = jnp.maximum(jnp.minimum(xi2, X2) - jnp.maximum(xi1, X1),
                                     0.0)
                    ih = jnp.maximum(jnp.minimum(yi2, Y2) - jnp.maximum(yi1, Y1),
                                     0.0)
                    inter = iw * ih
                    iou = inter / (ai + A - inter + 1e-8)
                    cm = jnp.logical_and(iou > 0.5, (li + 16 * v) > _bc(i, jnp.int32))
                    bits = _tree_sum(jnp.where(cm, one_shift, 0))
                    pos = i * (B // 16) + v
                    wd = obits[pl.ds(pos, 16)]
                    obits[pl.ds(pos, 16)] = jnp.where(li == 0, bits, wd)
                    return 0
                lax.fori_loop(w0, B // 16, col_body, 0)
                return 0
            lax.fori_loop(0, K, row_body, 0)

            suppw[pl.ds(0, 16)] = zi
            suppw[pl.ds(16, 16)] = zi
            suppw[pl.ds(32, 16)] = zi
            keepw[pl.ds(0, 16)] = zi
            keepw[pl.ds(16, 16)] = zi
            keepw[pl.ds(32, 16)] = zi

            def nms_body(i, _):
                w = i >> 4
                bpos = i & 15
                supv = suppw[pl.ds(w, 16)]
                sup = (supv[0] >> bpos) & 1
                live = jnp.logical_and(sup == 0, _sload(tops, i) > 0.0)
                kv = keepw[pl.ds(w, 16)]
                nkw = kv[0] | (jnp.where(live, 1, 0) << bpos)
                keepw[pl.ds(w, 16)] = jnp.where(li == 0, _bc(nkw, jnp.int32),
                                                kv)

                @pl.when(live)
                def _():
                    ob0 = obits[pl.ds(i * (B // 16) + w, 16)]
                    suppw[pl.ds(w, 16)] = suppw[pl.ds(w, 16)] | ob0
                    ob1v = obits[pl.ds(i * (B // 16) + w + 16, 16)]
                    suppw[pl.ds(w + 16, 16)] = suppw[pl.ds(w + 16, 16)] | ob1v
                return 0
            lax.fori_loop(0, K, nms_body, 0)

            for v in range(B // 16):
                kvec = (_bc(_sload(keepw, v), jnp.int32) >> li) & 1
                sv = tops[pl.ds(16 * v, 16)]
                colid = li + 16 * v
                outv = jnp.where(colid < K,
                                 jnp.where(kvec == 1, sv, jnp.float32(-1.0)),
                                 jnp.float32(-0.0))
                outbuf[pl.ds(16 * v, 16)] = outv

            pltpu.sync_copy(outbuf, outS.at[pl.ds(c * B, B)])
            pltpu.sync_copy(bx1.at[pl.ds(0, B)], ob1.at[pl.ds(c * B, B)])
            pltpu.sync_copy(by1.at[pl.ds(0, B)], ob2.at[pl.ds(c * B, B)])
            pltpu.sync_copy(bx2.at[pl.ds(0, B)], ob3.at[pl.ds(c * B, B)])
            pltpu.sync_copy(by2.at[pl.ds(0, B)], ob4.at[pl.ds(c * B, B)])
        return 0

    lax.fori_loop(0, 3, class_body, 0)


_sc_main = functools.partial(
    pl.kernel,
    out_type=[jax.ShapeDtypeStruct((C * B,), jnp.float32)] * 5,
    mesh=plsc.VectorSubcoreMesh(core_axis_name="c", subcore_axis_name="s"),
    scratch_types=[
        pltpu.VMEM((N,), jnp.float32),
        pltpu.VMEM((N,), jnp.int32),
        pltpu.VMEM((NBLK + 16,), jnp.int32),
        pltpu.VMEM((SORTN,), jnp.int32),
        pltpu.VMEM((SORTN,), jnp.int32),
        pltpu.VMEM((3, 128), jnp.int32),
        pltpu.VMEM((320,), jnp.float32),
        pltpu.VMEM((BG,), jnp.float32),
        pltpu.VMEM((BG,), jnp.float32),
        pltpu.VMEM((BG,), jnp.float32),
        pltpu.VMEM((BG,), jnp.float32),
        pltpu.VMEM((BG,), jnp.float32),
        pltpu.VMEM((K * (B // 16) + 64,), jnp.int32),
        pltpu.VMEM((64,), jnp.int32),
        pltpu.VMEM((64,), jnp.int32),
        pltpu.VMEM((B,), jnp.float32),
        pltpu.VMEM((16,), jnp.int32),
        pltpu.VMEM((16,), jnp.int32),
        pltpu.VMEM((16,), jnp.int32),
        pltpu.VMEM((FBUF,), jnp.int32),
        pltpu.VMEM((FBUF,), jnp.int32),
        pltpu.VMEM((16,), jnp.int32),
        pltpu.SemaphoreType.DMA,
    ],
)(_sc_main_body)


def _sc_merge_body(flatS, fb1, fb2, fb3, fb4,
                   fscore, fidx, fo1, fo2, fo3, fo4,
                   scores_v, ksbuf, bmax, sortK, sortI, candI2, tops,
                   bx1, by1, bx2, by2, cntbuf, gob, eob, fbK, fbI, fob, sem):
    wid = lax.axis_index("s") * 2 + lax.axis_index("c")

    @pl.when(wid == 0)
    def _():
        pltpu.sync_copy(flatS, scores_v)
        _fill_ks(scores_v, ksbuf, bmax, NBLK_M, False)
        _bisect_select(ksbuf, bmax, NBLK_M, NVEC_M, sortK, sortI, cntbuf,
                       gob, eob, fbK, fbI, fob)
        _bitonic_sort_512(sortK, sortI)
        _extract_sorted(sortK, sortI, candI2, tops)
        _gather_boxes((fb1, fb2, fb3, fb4), candI2,
                      (bx1, by1, bx2, by2), sem)
        pltpu.sync_copy(tops.at[pl.ds(0, B)], fscore)
        pltpu.sync_copy(candI2.at[0], fidx.at[pl.ds(0, 128)])
        pltpu.sync_copy(candI2.at[1], fidx.at[pl.ds(128, 128)])
        pltpu.sync_copy(candI2.at[2], fidx.at[pl.ds(256, 128)])
        pltpu.sync_copy(bx1.at[pl.ds(0, B)], fo1)
        pltpu.sync_copy(by1.at[pl.ds(0, B)], fo2)
        pltpu.sync_copy(bx2.at[pl.ds(0, B)], fo3)
        pltpu.sync_copy(by2.at[pl.ds(0, B)], fo4)


_sc_merge = functools.partial(
    pl.kernel,
    out_type=[jax.ShapeDtypeStruct((B,), jnp.float32),
              jax.ShapeDtypeStruct((BG,), jnp.int32)] +
             [jax.ShapeDtypeStruct((B,), jnp.float32)] * 4,
    mesh=plsc.VectorSubcoreMesh(core_axis_name="c", subcore_axis_name="s"),
    scratch_types=[
        pltpu.VMEM((C * B,), jnp.float32),
        pltpu.VMEM((C * B,), jnp.int32),
        pltpu.VMEM((NBLK_M + 16,), jnp.int32),
        pltpu.VMEM((SORTN,), jnp.int32),
        pltpu.VMEM((SORTN,), jnp.int32),
        pltpu.VMEM((3, 128), jnp.int32),
        pltpu.VMEM((320,), jnp.float32),
        pltpu.VMEM((BG,), jnp.float32),
        pltpu.VMEM((BG,), jnp.float32),
        pltpu.VMEM((BG,), jnp.float32),
        pltpu.VMEM((BG,), jnp.float32),
        pltpu.VMEM((16,), jnp.int32),
        pltpu.VMEM((16,), jnp.int32),
        pltpu.VMEM((16,), jnp.int32),
        pltpu.VMEM((FBUF,), jnp.int32),
        pltpu.VMEM((FBUF,), jnp.int32),
        pltpu.VMEM((16,), jnp.int32),
        pltpu.SemaphoreType.DMA,
    ],
)(_sc_merge_body)


@jax.jit
def kernel(classifications, regressions, anchors):
    x1p, y1p, x2p, y2p = _decode_boxes(anchors[0], regressions[0])
    scoresT = jnp.transpose(classifications[0]).reshape(-1)
    outS, ob1, ob2, ob3, ob4 = _sc_main(scoresT, x1p, y1p, x2p, y2p)
    fs, fidx, f1, f2, f3, f4 = _sc_merge(outS, ob1, ob2, ob3, ob4)
    final_scores = fs[:K]
    final_labels = (fidx[:K] // B).astype(jnp.int32)
    final_boxes = jnp.stack([f1[:K], f2[:K], f3[:K], f4[:K]], axis=-1)
    return (final_scores, final_labels, final_boxes)


# speedup vs baseline: 3.6399x; 1.0047x over previous
import functools
import numpy as np
import jax
import jax.numpy as jnp
from jax import lax
from jax.experimental import pallas as pl
from jax.experimental.pallas import tpu as pltpu
from jax.experimental.pallas import tpu_sc as plsc

N = 20000
C = 80
K = 300
CLS_THRES = 0.05
IMG_H = 640.0
IMG_W = 640.0

NPAD = 20096
ROWS = 157
LANES = 128

NW = 32
B = 304
BG = 384
SORTN = 512
CAP = 496
FCAP = 4064
FBUF = 4112
BLK = 10
NVEC = N // 16
NBLK = NVEC // BLK
NVEC_M = (C * B) // 16
NBLK_M = NVEC_M // BLK
SIGN = np.uint32(0x80000000)
IMIN = np.int32(-2**31)


def _li():
    return lax.iota(jnp.int32, 16)


def _sload(ref, i):
    return ref[pl.ds(i, 16)][0]


def _bc(x, dtype):
    return jnp.full((16,), x, dtype=dtype)


_GDN = lax.GatherDimensionNumbers(
    offset_dims=(), collapsed_slice_dims=(0,), start_index_map=(0,))


def _laneperm(x, idx):
    return lax.gather(x, idx.reshape(16, 1), _GDN, (1,),
                      mode=lax.GatherScatterMode.PROMISE_IN_BOUNDS)


def _tree_sum(x):
    li = _li()
    for d in (1, 2, 4, 8):
        x = x + _laneperm(x, li ^ d)
    return x


def _tree_max(x):
    li = _li()
    for d in (1, 2, 4, 8):
        x = jnp.maximum(x, _laneperm(x, li ^ d))
    return x


def _tree_min(x):
    li = _li()
    for d in (1, 2, 4, 8):
        x = jnp.minimum(x, _laneperm(x, li ^ d))
    return x


def _incl_prefix(m_i32):
    li = _li()
    x = m_i32
    for d in (1, 2, 4, 8):
        sh = _laneperm(x, jnp.maximum(li - d, 0))
        x = x + jnp.where(li >= d, sh, 0)
    return x


def _compact_src(incl):
    li = _li()
    pos = jnp.zeros((16,), jnp.int32)
    tgt = li + 1
    for b in (8, 4, 2, 1):
        g = _laneperm(incl, pos + (b - 1))
        pos = jnp.where(g < tgt, pos + b, pos)
    return pos


def _uk_to_ks(thr_u32):
    return lax.convert_element_type(thr_u32 ^ SIGN, jnp.int32)


def _decode_body(ax1, ay1, ax2, ay2, dx, dy, dw, dh, x1o, y1o, x2o, y2o):
    wa = ax2[...] - ax1[...]
    ha = ay2[...] - ay1[...]
    cxa = ax1[...] + 0.5 * wa
    cya = ay1[...] + 0.5 * ha
    cx = cxa + dx[...] * 0.1 * wa
    cy = cya + dy[...] * 0.1 * ha
    w = jnp.exp(dw[...] * 0.2) * wa
    h = jnp.exp(dh[...] * 0.2) * ha
    x1o[...] = jnp.clip(cx - 0.5 * w, 0.0, IMG_W)
    y1o[...] = jnp.clip(cy - 0.5 * h, 0.0, IMG_H)
    x2o[...] = jnp.clip(cx + 0.5 * w, 0.0, IMG_W)
    y2o[...] = jnp.clip(cy + 0.5 * h, 0.0, IMG_H)


def _decode_boxes(anchors, regressions):
    def planar(a):
        pads = jnp.zeros((NPAD - N,), a.dtype)
        return [jnp.concatenate([a[:, i], pads]).reshape(ROWS, LANES)
                for i in range(4)]

    ins = planar(anchors) + planar(regressions)
    outs = pl.pallas_call(
        _decode_body,
        out_shape=[jax.ShapeDtypeStruct((ROWS, LANES), jnp.float32)] * 4,
    )(*ins)
    return [o.reshape(NPAD) for o in outs]


def _fill_ks(src_ref, ks_ref, bmax_ref, nblk, mask_thres):
    li = _li()

    def blk(bi, _):
        base = bi * BLK
        acc = _bc(IMIN, jnp.int32)
        for jj in range(BLK):
            o = pl.ds(16 * (base + jj), 16)
            s = src_ref[o]
            if mask_thres:
                s = jnp.where(s > CLS_THRES, s, jnp.float32(-1.0))
            ks = lax.bitcast_convert_type(s, jnp.int32)
            ks_ref[o] = ks
            acc = jnp.maximum(acc, ks)
        bm = _tree_max(acc)
        wd = bmax_ref[pl.ds(bi, 16)]
        bmax_ref[pl.ds(bi, 16)] = jnp.where(li == 0, bm, wd)
        return 0
    lax.fori_loop(0, nblk, blk, 0)


def _count_ge(ks_ref, bmax_ref, nblk, thr_ks, cntbuf):
    cntbuf[pl.ds(0, 16)] = jnp.zeros((16,), jnp.int32)
    thrv = _bc(thr_ks, jnp.int32)

    def blk(bi, _):
        bm = _sload(bmax_ref, bi)

        @pl.when(bm >= thr_ks)
        def _():
            base = bi * BLK
            acc = jnp.zeros((16,), jnp.int32)
            for jj in range(BLK):
                u = ks_ref[pl.ds(16 * (base + jj), 16)]
                acc = acc + jnp.where(u >= thrv, 1, 0)
            cntbuf[pl.ds(0, 16)] = cntbuf[pl.ds(0, 16)] + acc
        return 0
    lax.fori_loop(0, nblk, blk, 0)


def _append(dstK, dstI, offbuf, ks, idxv, mask, cap=SORTN):
    li = _li()
    incl = _incl_prefix(jnp.where(mask, 1, 0))
    cnt = lax.rev(incl, (0,))[0]
    src = _compact_src(incl)
    ck = _laneperm(ks, src)
    ci = _laneperm(idxv, src)

    @pl.when(cnt > 0)
    def _():
        cntv = _bc(cnt, jnp.int32)
        offv = offbuf[pl.ds(0, 16)]
        off = jnp.minimum(offv[0], cap - 16)
        sel = li < cntv
        dstK[pl.ds(off, 16)] = jnp.where(sel, ck, dstK[pl.ds(off, 16)])
        dstI[pl.ds(off, 16)] = jnp.where(sel, ci, dstI[pl.ds(off, 16)])
        offbuf[pl.ds(0, 16)] = offv + cntv


def _ks_to_u(ks):
    return lax.convert_element_type(ks, jnp.uint32) ^ SIGN


def _bisect_core(count_fn, cntbuf, lo0, hi0):
    def step(_s, c):
        lo, hi, done_i, GTH, EQLO, capped_i = c
        active = done_i == 0
        mid = lo + np.uint32(1) + lax.shift_right_logical(
            hi - lo - np.uint32(1), np.uint32(1))
        mid_ks = _uk_to_ks(mid)

        @pl.when(active)
        def _():
            count_fn(mid_ks)

        cnt = _tree_sum(cntbuf[pl.ds(0, 16)])[0]
        geK = cnt >= K
        hitw = jnp.logical_and(active, jnp.logical_and(geK, cnt <= CAP))
        lo2 = jnp.where(jnp.logical_and(active, geK), mid, lo)
        hi2 = jnp.where(jnp.logical_and(active, jnp.logical_not(geK)),
                        mid - np.uint32(1), hi)
        conv = jnp.logical_and(jnp.logical_and(active,
                                               jnp.logical_not(hitw)),
                               lo2 == hi2)
        GTH2 = jnp.where(hitw, mid - np.uint32(1),
                         jnp.where(conv, lo2, GTH))
        EQLO2 = jnp.where(hitw, mid, jnp.where(conv, lo2, EQLO))
        capped2 = jnp.where(conv, 1, capped_i)
        done2 = jnp.where(jnp.logical_or(hitw, conv), 1, done_i)
        return (lo2, hi2, done2, GTH2, EQLO2, capped2)

    _, _, _, GTH, EQLO, capped_i = lax.fori_loop(
        0, 33, step, (lo0, hi0, jnp.int32(0), lo0, lo0, jnp.int32(0)))

    m1_ks = _uk_to_ks(GTH + np.uint32(1))

    @pl.when(capped_i == 1)
    def _():
        count_fn(m1_ks)

    m = jnp.where(capped_i == 1, _tree_sum(cntbuf[pl.ds(0, 16)])[0], 0)
    return GTH, EQLO, capped_i, m


def _compact_pair(dstK, dstI, gob, eob, u, idxv, gthv, eqlov):
    _append(dstK, dstI, gob, u, idxv, u > gthv)
    eq = jnp.logical_and(u >= eqlov, u <= gthv)
    ape = _sload(eob, 0) < K
    apev = _bc(jnp.where(ape, 1, 0), jnp.int32)
    eqm = (jnp.where(eq, 1, 0) & apev) == 1
    _append(dstK, dstI, eob, u, idxv, eqm)


def _bisect_select(ks_ref, bmax_ref, nblk, nvec, sortK_ref, sortI_ref,
                   cntbuf, gob, eob, fbK, fbI, fob):
    li = _li()
    zk = _bc(IMIN, jnp.int32)
    zi = jnp.zeros((16,), jnp.int32)

    def zbody(v, _):
        sortK_ref[pl.ds(16 * v, 16)] = zk
        sortI_ref[pl.ds(16 * v, 16)] = zi
        return 0
    lax.fori_loop(0, SORTN // 16, zbody, 0)

    accmin = _bc(np.int32(2**31 - 1), jnp.int32)
    accmax = _bc(IMIN, jnp.int32)
    for v in range((nblk + 15) // 16):
        w = bmax_ref[pl.ds(16 * v, 16)]
        valid = (li + 16 * v) < nblk
        accmin = jnp.minimum(accmin, jnp.where(valid, w, np.int32(2**31 - 1)))
        accmax = jnp.maximum(accmax, jnp.where(valid, w, IMIN))
    t0k = lax.rev(_tree_min(accmin), (0,))[0]
    gmaxu = _ks_to_u(lax.rev(_tree_max(accmax), (0,))[0])

    _count_ge(ks_ref, bmax_ref, nblk, t0k, cntbuf)
    cnt0 = _tree_sum(cntbuf[pl.ds(0, 16)])[0]
    fast = jnp.logical_and(cnt0 >= K, cnt0 <= FCAP)

    @pl.when(fast)
    def _():
        fob[pl.ds(0, 16)] = zi
        t0v = _bc(t0k, jnp.int32)

        def fblk(j, _):
            u = ks_ref[pl.ds(16 * j, 16)]
            _append(fbK, fbI, fob, u, li + 16 * j, u >= t0v, cap=FBUF)
            return 0
        lax.fori_loop(0, nvec, fblk, 0)
        flen = _sload(fob, 0)
        fbK[pl.ds(flen, 16)] = zk
        fbI[pl.ds(flen, 16)] = zi
        nvb = (flen + 15) >> 4

        def bcount(thr_ks):
            thrv = _bc(thr_ks, jnp.int32)

            def b(j, a):
                return a + jnp.where(fbK[pl.ds(16 * j, 16)] >= thrv, 1, 0)
            cntbuf[pl.ds(0, 16)] = lax.fori_loop(
                0, nvb, b, jnp.zeros((16,), jnp.int32))

        GTH, EQLO, capped_i, m = _bisect_core(bcount, cntbuf,
                                              _ks_to_u(t0k), gmaxu)
        gob[pl.ds(0, 16)] = zi
        eob[pl.ds(0, 16)] = _bc(m, jnp.int32)
        gthv = _bc(_uk_to_ks(GTH), jnp.int32)
        eqlov = _bc(_uk_to_ks(EQLO), jnp.int32)

        def cb(j, _):
            u = fbK[pl.ds(16 * j, 16)]
            iv = fbI[pl.ds(16 * j, 16)]
            _compact_pair(sortK_ref, sortI_ref, gob, eob, u, iv, gthv, eqlov)
            return 0
        lax.fori_loop(0, nvb, cb, 0)

    @pl.when(jnp.logical_not(fast))
    def _():
        def fcount(thr_ks):
            _count_ge(ks_ref, bmax_ref, nblk, thr_ks, cntbuf)

        GTH, EQLO, capped_i, m = _bisect_core(
            fcount, cntbuf, np.uint32(0x3F800000), gmaxu)
        gob[pl.ds(0, 16)] = zi
        eob[pl.ds(0, 16)] = _bc(m, jnp.int32)
        eqlo_ks = _uk_to_ks(EQLO)
        gthv = _bc(_uk_to_ks(GTH), jnp.int32)
        eqlov = _bc(eqlo_ks, jnp.int32)

        def cblk(bi, _):
            bm = _sload(bmax_ref, bi)

            @pl.when(bm >= eqlo_ks)
            def _():
                base = bi * BLK
                for jj in range(BLK):
                    u = ks_ref[pl.ds(16 * (base + jj), 16)]
                    idxv = li + 16 * (base + jj)
                    _compact_pair(sortK_ref, sortI_ref, gob, eob, u, idxv,
                                  gthv, eqlov)
            return 0
        lax.fori_loop(0, nblk, cblk, 0)


def _bitonic_sort_512(sortK_ref, sortI_ref):
    li = _li()

    def xlayer(kk, j, logkk, logj):
        jr = j // 16
        logjr = logj - 4

        def body(i, _):
            r1 = (i & (jr - 1)) | ((i >> logjr) << (logjr + 1))
            dscv = _bc((r1 >> (logkk - 4)) & 1, jnp.int32)
            o1 = 16 * r1
            o2 = o1 + j
            K1 = sortK_ref[pl.ds(o1, 16)]
            K2 = sortK_ref[pl.ds(o2, 16)]
            I1 = sortI_ref[pl.ds(o1, 16)]
            I2 = sortI_ref[pl.ds(o2, 16)]
            better1 = jnp.logical_or(K1 > K2,
                                     jnp.logical_and(K1 == K2, I1 < I2))
            keep = (jnp.where(better1, 1, 0) ^ dscv) == 1
            sortK_ref[pl.ds(o1, 16)] = jnp.where(keep, K1, K2)
            sortK_ref[pl.ds(o2, 16)] = jnp.where(keep, K2, K1)
            sortI_ref[pl.ds(o1, 16)] = jnp.where(keep, I1, I2)
            sortI_ref[pl.ds(o2, 16)] = jnp.where(keep, I2, I1)
            return 0
        lax.fori_loop(0, 16, body, 0)

    def llayer(kk, jl, logkk):
        pidx = li ^ jl
        upper = (li & jl) != 0

        upv = jnp.where(upper, 1, 0)

        def body(r, _):
            o = 16 * r
            Kv = sortK_ref[pl.ds(o, 16)]
            Iv = sortI_ref[pl.ds(o, 16)]
            pK = _laneperm(Kv, pidx)
            pI = _laneperm(Iv, pidx)
            if logkk < 4:
                flipv = ((li >> logkk) & 1) ^ upv
            else:
                flipv = _bc((r >> (logkk - 4)) & 1, jnp.int32) ^ upv
            better = jnp.logical_or(Kv > pK,
                                    jnp.logical_and(Kv == pK, Iv < pI))
            keep = (jnp.where(better, 1, 0) ^ flipv) == 1
            sortK_ref[pl.ds(o, 16)] = jnp.where(keep, Kv, pK)
            sortI_ref[pl.ds(o, 16)] = jnp.where(keep, Iv, pI)
            return 0
        lax.fori_loop(0, 32, body, 0)

    kk = 2
    while kk <= SORTN:
        logkk = kk.bit_length() - 1
        j = kk // 2
        while j >= 1:
            if j >= 16:
                xlayer(kk, j, logkk, j.bit_length() - 1)
            else:
                llayer(kk, j, logkk)
            j //= 2
        kk *= 2


def _extract_sorted(sortK_ref, sortI_ref, candI2_ref, tops_ref):
    for o in range(BG // 16):
        ii = sortI_ref[pl.ds(16 * o, 16)]
        candI2_ref[o // 8, pl.ds((o % 8) * 16, 16)] = ii
        if o < B // 16:
            ki = sortK_ref[pl.ds(16 * o, 16)]
            tops_ref[pl.ds(16 * o, 16)] = lax.bitcast_convert_type(
                ki, jnp.float32)


def _gather_boxes(planes, candI2_ref, dsts, sem):
    cps = []
    for g in range(3):
        for plane, dst in zip(planes, dsts):
            cp = pltpu.make_async_copy(plane.at[candI2_ref.at[g]],
                                       dst.at[pl.ds(128 * g, 128)], sem)
            cp.start()
            cps.append(cp)
    for cp in cps:
        cp.wait()


def _sc_main_body(scoresT, x1p, y1p, x2p, y2p,
                  outS, ob1, ob2, ob3, ob4,
                  scores_v, ksbuf, bmax, sortK, sortI, candI2, tops,
                  bx1, by1, bx2, by2, ar, obits, suppw, keepw, outbuf,
                  cntbuf, gob, eob, fbK, fbI, fob, sem):
    wid = lax.axis_index("s") * 2 + lax.axis_index("c")
    li = _li()
    one_shift = lax.shift_left(jnp.ones((16,), jnp.int32), li)
    zi = jnp.zeros((16,), jnp.int32)

    def class_body(t, _):
        c = wid + NW * t

        @pl.when(c < C)
        def _():
            pltpu.sync_copy(scoresT.at[pl.ds(c * N, N)], scores_v)
            _fill_ks(scores_v, ksbuf, bmax, NBLK, True)
            _bisect_select(ksbuf, bmax, NBLK, NVEC, sortK, sortI, cntbuf,
                           gob, eob, fbK, fbI, fob)
            _bitonic_sort_512(sortK, sortI)
            _extract_sorted(sortK, sortI, candI2, tops)
            _gather_boxes((x1p, y1p, x2p, y2p), candI2,
                          (bx1, by1, bx2, by2), sem)

            def area_body(v, _):
                o = pl.ds(16 * v, 16)
                ar[o] = (bx2[o] - bx1[o]) * (by2[o] - by1[o])
                return 0
            lax.fori_loop(0, B // 16, area_body, 0)

            def row_body(i):
                w0 = i >> 4
                xi1 = _sload(bx1, i)
                yi1 = _sload(by1, i)
                xi2 = _sload(bx2, i)
                yi2 = _sload(by2, i)
                ai = _sload(ar, i)

                def col_body(v, _):
                    o = pl.ds(16 * v, 16)
                    X1 = bx1[o]
                    Y1 = by1[o]
                    X2 = bx2[o]
                    Y2 = by2[o]
                    A = ar[o]
                    iw = jnp.maximum(jnp.minimum(xi2, X2) - jnp.maximum(xi1, X1),
                                     0.0)
                    ih = jnp.maximum(jnp.minimum(yi2, Y2) - jnp.maximum(yi1, Y1),
                                     0.0)
                    inter = iw * ih
                    iou = inter / (ai + A - inter + 1e-8)
                    cm = jnp.logical_and(iou > 0.5, (li + 16 * v) > _bc(i, jnp.int32))
                    bits = _tree_sum(jnp.where(cm, one_shift, 0))
                    pos = i * (B // 16) + v
                    wd = obits[pl.ds(pos, 16)]
                    obits[pl.ds(pos, 16)] = jnp.where(li == 0, bits, wd)
                    return 0
                lax.fori_loop(w0, B // 16, col_body, 0)

            plsc.parallel_loop(0, K, 1, unroll=2)(row_body)

            suppw[pl.ds(0, 16)] = zi
            suppw[pl.ds(16, 16)] = zi
            suppw[pl.ds(32, 16)] = zi
            keepw[pl.ds(0, 16)] = zi
            keepw[pl.ds(16, 16)] = zi
            keepw[pl.ds(32, 16)] = zi

            def nms_body(i, _):
                w = i >> 4
                bpos = i & 15
                supv = suppw[pl.ds(w, 16)]
                sup = (supv[0] >> bpos) & 1
                live = jnp.logical_and(sup == 0, _sload(tops, i) > 0.0)
                kv = keepw[pl.ds(w, 16)]
                nkw = kv[0] | (jnp.where(live, 1, 0) << bpos)
                keepw[pl.ds(w, 16)] = jnp.where(li == 0, _bc(nkw, jnp.int32),
                                                kv)

                @pl.when(live)
                def _():
                    ob0 = obits[pl.ds(i * (B // 16) + w, 16)]
                    suppw[pl.ds(w, 16)] = suppw[pl.ds(w, 16)] | ob0
                    ob1v = obits[pl.ds(i * (B // 16) + w + 16, 16)]
                    suppw[pl.ds(w + 16, 16)] = suppw[pl.ds(w + 16, 16)] | ob1v
                return 0
            lax.fori_loop(0, K, nms_body, 0)

            for v in range(B // 16):
                kvec = (_bc(_sload(keepw, v), jnp.int32) >> li) & 1
                sv = tops[pl.ds(16 * v, 16)]
                colid = li + 16 * v
                outv = jnp.where(colid < K,
                                 jnp.where(kvec == 1, sv, jnp.float32(-1.0)),
                                 jnp.float32(-0.0))
                outbuf[pl.ds(16 * v, 16)] = outv

            pltpu.sync_copy(outbuf, outS.at[pl.ds(c * B, B)])
            pltpu.sync_copy(bx1.at[pl.ds(0, B)], ob1.at[pl.ds(c * B, B)])
            pltpu.sync_copy(by1.at[pl.ds(0, B)], ob2.at[pl.ds(c * B, B)])
            pltpu.sync_copy(bx2.at[pl.ds(0, B)], ob3.at[pl.ds(c * B, B)])
            pltpu.sync_copy(by2.at[pl.ds(0, B)], ob4.at[pl.ds(c * B, B)])
        return 0

    lax.fori_loop(0, 3, class_body, 0)


_sc_main = functools.partial(
    pl.kernel,
    out_type=[jax.ShapeDtypeStruct((C * B,), jnp.float32)] * 5,
    mesh=plsc.VectorSubcoreMesh(core_axis_name="c", subcore_axis_name="s"),
    scratch_types=[
        pltpu.VMEM((N,), jnp.float32),
        pltpu.VMEM((N,), jnp.int32),
        pltpu.VMEM((NBLK + 16,), jnp.int32),
        pltpu.VMEM((SORTN,), jnp.int32),
        pltpu.VMEM((SORTN,), jnp.int32),
        pltpu.VMEM((3, 128), jnp.int32),
        pltpu.VMEM((320,), jnp.float32),
        pltpu.VMEM((BG,), jnp.float32),
        pltpu.VMEM((BG,), jnp.float32),
        pltpu.VMEM((BG,), jnp.float32),
        pltpu.VMEM((BG,), jnp.float32),
        pltpu.VMEM((BG,), jnp.float32),
        pltpu.VMEM((K * (B // 16) + 64,), jnp.int32),
        pltpu.VMEM((64,), jnp.int32),
        pltpu.VMEM((64,), jnp.int32),
        pltpu.VMEM((B,), jnp.float32),
        pltpu.VMEM((16,), jnp.int32),
        pltpu.VMEM((16,), jnp.int32),
        pltpu.VMEM((16,), jnp.int32),
        pltpu.VMEM((FBUF,), jnp.int32),
        pltpu.VMEM((FBUF,), jnp.int32),
        pltpu.VMEM((16,), jnp.int32),
        pltpu.SemaphoreType.DMA,
    ],
)(_sc_main_body)


def _sc_merge_body(flatS, fb1, fb2, fb3, fb4,
                   fscore, fidx, fo1, fo2, fo3, fo4,
                   scores_v, ksbuf, bmax, sortK, sortI, candI2, tops,
                   bx1, by1, bx2, by2, cntbuf, gob, eob, fbK, fbI, fob, sem):
    wid = lax.axis_index("s") * 2 + lax.axis_index("c")

    @pl.when(wid == 0)
    def _():
        pltpu.sync_copy(flatS, scores_v)
        _fill_ks(scores_v, ksbuf, bmax, NBLK_M, False)
        _bisect_select(ksbuf, bmax, NBLK_M, NVEC_M, sortK, sortI, cntbuf,
                       gob, eob, fbK, fbI, fob)
        _bitonic_sort_512(sortK, sortI)
        _extract_sorted(sortK, sortI, candI2, tops)
        _gather_boxes((fb1, fb2, fb3, fb4), candI2,
                      (bx1, by1, bx2, by2), sem)
        pltpu.sync_copy(tops.at[pl.ds(0, B)], fscore)
        pltpu.sync_copy(candI2.at[0], fidx.at[pl.ds(0, 128)])
        pltpu.sync_copy(candI2.at[1], fidx.at[pl.ds(128, 128)])
        pltpu.sync_copy(candI2.at[2], fidx.at[pl.ds(256, 128)])
        pltpu.sync_copy(bx1.at[pl.ds(0, B)], fo1)
        pltpu.sync_copy(by1.at[pl.ds(0, B)], fo2)
        pltpu.sync_copy(bx2.at[pl.ds(0, B)], fo3)
        pltpu.sync_copy(by2.at[pl.ds(0, B)], fo4)


_sc_merge = functools.partial(
    pl.kernel,
    out_type=[jax.ShapeDtypeStruct((B,), jnp.float32),
              jax.ShapeDtypeStruct((BG,), jnp.int32)] +
             [jax.ShapeDtypeStruct((B,), jnp.float32)] * 4,
    mesh=plsc.VectorSubcoreMesh(core_axis_name="c", subcore_axis_name="s"),
    scratch_types=[
        pltpu.VMEM((C * B,), jnp.float32),
        pltpu.VMEM((C * B,), jnp.int32),
        pltpu.VMEM((NBLK_M + 16,), jnp.int32),
        pltpu.VMEM((SORTN,), jnp.int32),
        pltpu.VMEM((SORTN,), jnp.int32),
        pltpu.VMEM((3, 128), jnp.int32),
        pltpu.VMEM((320,), jnp.float32),
        pltpu.VMEM((BG,), jnp.float32),
        pltpu.VMEM((BG,), jnp.float32),
        pltpu.VMEM((BG,), jnp.float32),
        pltpu.VMEM((BG,), jnp.float32),
        pltpu.VMEM((16,), jnp.int32),
        pltpu.VMEM((16,), jnp.int32),
        pltpu.VMEM((16,), jnp.int32),
        pltpu.VMEM((FBUF,), jnp.int32),
        pltpu.VMEM((FBUF,), jnp.int32),
        pltpu.VMEM((16,), jnp.int32),
        pltpu.SemaphoreType.DMA,
    ],
)(_sc_merge_body)


@jax.jit
def kernel(classifications, regressions, anchors):
    x1p, y1p, x2p, y2p = _decode_boxes(anchors[0], regressions[0])
    scoresT = jnp.transpose(classifications[0]).reshape(-1)
    outS, ob1, ob2, ob3, ob4 = _sc_main(scoresT, x1p, y1p, x2p, y2p)
    fs, fidx, f1, f2, f3, f4 = _sc_merge(outS, ob1, ob2, ob3, ob4)
    final_scores = fs[:K]
    final_labels = (fidx[:K] // B).astype(jnp.int32)
    final_boxes = jnp.stack([f1[:K], f2[:K], f3[:K], f4[:K]], axis=-1)
    return (final_scores, final_labels, final_boxes)


# speedup vs baseline: 3.7716x; 1.0362x over previous
import functools
import numpy as np
import jax
import jax.numpy as jnp
from jax import lax
from jax.experimental import pallas as pl
from jax.experimental.pallas import tpu as pltpu
from jax.experimental.pallas import tpu_sc as plsc

N = 20000
C = 80
K = 300
CLS_THRES = 0.05
IMG_H = 640.0
IMG_W = 640.0

NPAD = 20096
ROWS = 157
LANES = 128

NW = 32
B = 304
BG = 384
SORTN = 512
CAP = 496
FCAP = 4064
FBUF = 4112
BLK = 10
NVEC = N // 16
NBLK = NVEC // BLK
NVEC_M = (C * B) // 16
NBLK_M = NVEC_M // BLK
SIGN = np.uint32(0x80000000)
IMIN = np.int32(-2**31)


def _li():
    return lax.iota(jnp.int32, 16)


def _sload(ref, i):
    return ref[pl.ds(i, 16)][0]


def _bc(x, dtype):
    return jnp.full((16,), x, dtype=dtype)


_GDN = lax.GatherDimensionNumbers(
    offset_dims=(), collapsed_slice_dims=(0,), start_index_map=(0,))


def _laneperm(x, idx):
    return lax.gather(x, idx.reshape(16, 1), _GDN, (1,),
                      mode=lax.GatherScatterMode.PROMISE_IN_BOUNDS)


def _tree_sum(x):
    li = _li()
    for d in (1, 2, 4, 8):
        x = x + _laneperm(x, li ^ d)
    return x


def _tree_max(x):
    li = _li()
    for d in (1, 2, 4, 8):
        x = jnp.maximum(x, _laneperm(x, li ^ d))
    return x


def _tree_min(x):
    li = _li()
    for d in (1, 2, 4, 8):
        x = jnp.minimum(x, _laneperm(x, li ^ d))
    return x


def _incl_prefix(m_i32):
    li = _li()
    x = m_i32
    for d in (1, 2, 4, 8):
        sh = _laneperm(x, jnp.maximum(li - d, 0))
        x = x + jnp.where(li >= d, sh, 0)
    return x


def _compact_src(incl):
    li = _li()
    pos = jnp.zeros((16,), jnp.int32)
    tgt = li + 1
    for b in (8, 4, 2, 1):
        g = _laneperm(incl, pos + (b - 1))
        pos = jnp.where(g < tgt, pos + b, pos)
    return pos


def _uk_to_ks(thr_u32):
    return lax.convert_element_type(thr_u32 ^ SIGN, jnp.int32)


def _decode_body(ax1, ay1, ax2, ay2, dx, dy, dw, dh, x1o, y1o, x2o, y2o):
    wa = ax2[...] - ax1[...]
    ha = ay2[...] - ay1[...]
    cxa = ax1[...] + 0.5 * wa
    cya = ay1[...] + 0.5 * ha
    cx = cxa + dx[...] * 0.1 * wa
    cy = cya + dy[...] * 0.1 * ha
    w = jnp.exp(dw[...] * 0.2) * wa
    h = jnp.exp(dh[...] * 0.2) * ha
    x1o[...] = jnp.clip(cx - 0.5 * w, 0.0, IMG_W)
    y1o[...] = jnp.clip(cy - 0.5 * h, 0.0, IMG_H)
    x2o[...] = jnp.clip(cx + 0.5 * w, 0.0, IMG_W)
    y2o[...] = jnp.clip(cy + 0.5 * h, 0.0, IMG_H)


def _decode_boxes(anchors, regressions):
    def planar(a):
        pads = jnp.zeros((NPAD - N,), a.dtype)
        return [jnp.concatenate([a[:, i], pads]).reshape(ROWS, LANES)
                for i in range(4)]

    ins = planar(anchors) + planar(regressions)
    outs = pl.pallas_call(
        _decode_body,
        out_shape=[jax.ShapeDtypeStruct((ROWS, LANES), jnp.float32)] * 4,
    )(*ins)
    return [o.reshape(NPAD) for o in outs]


def _ld_ks(src_ref, off, mode):
    v = src_ref[pl.ds(off, 16)]
    if mode == "ks":
        return v
    if mode == "f32mask":
        v = jnp.where(v > CLS_THRES, v, jnp.float32(-1.0))
    return lax.bitcast_convert_type(v, jnp.int32)


def _fill_ks(src_ref, ks_ref, bmax_ref, nblk, mode):
    li = _li()

    def blk(bi, _):
        base = bi * BLK
        acc = _bc(IMIN, jnp.int32)
        for jj in range(BLK):
            ks = _ld_ks(src_ref, 16 * (base + jj), mode)
            if ks_ref is not None:
                ks_ref[pl.ds(16 * (base + jj), 16)] = ks
            acc = jnp.maximum(acc, ks)
        bm = _tree_max(acc)
        wd = bmax_ref[pl.ds(bi, 16)]
        bmax_ref[pl.ds(bi, 16)] = jnp.where(li == 0, bm, wd)
        return 0
    lax.fori_loop(0, nblk, blk, 0)


def _count_ge(src_ref, mode, bmax_ref, nblk, thr_ks, cntbuf):
    cntbuf[pl.ds(0, 16)] = jnp.zeros((16,), jnp.int32)
    thrv = _bc(thr_ks, jnp.int32)

    def blk(bi, _):
        bm = _sload(bmax_ref, bi)

        @pl.when(bm >= thr_ks)
        def _():
            base = bi * BLK
            acc = jnp.zeros((16,), jnp.int32)
            for jj in range(BLK):
                u = _ld_ks(src_ref, 16 * (base + jj), mode)
                acc = acc + jnp.where(u >= thrv, 1, 0)
            cntbuf[pl.ds(0, 16)] = cntbuf[pl.ds(0, 16)] + acc
        return 0
    lax.fori_loop(0, nblk, blk, 0)


def _append(dstK, dstI, offbuf, ks, idxv, mask, cap=SORTN):
    li = _li()
    incl = _incl_prefix(jnp.where(mask, 1, 0))
    cnt = lax.rev(incl, (0,))[0]
    src = _compact_src(incl)
    ck = _laneperm(ks, src)
    ci = _laneperm(idxv, src)

    @pl.when(cnt > 0)
    def _():
        cntv = _bc(cnt, jnp.int32)
        offv = offbuf[pl.ds(0, 16)]
        off = jnp.minimum(offv[0], cap - 16)
        sel = li < cntv
        dstK[pl.ds(off, 16)] = jnp.where(sel, ck, dstK[pl.ds(off, 16)])
        dstI[pl.ds(off, 16)] = jnp.where(sel, ci, dstI[pl.ds(off, 16)])
        offbuf[pl.ds(0, 16)] = offv + cntv


def _ks_to_u(ks):
    return lax.convert_element_type(ks, jnp.uint32) ^ SIGN


def _bisect_core(count_fn, cntbuf, lo0, hi0):
    def step(_s, c):
        lo, hi, done_i, GTH, EQLO, capped_i = c
        active = done_i == 0
        mid = lo + np.uint32(1) + lax.shift_right_logical(
            hi - lo - np.uint32(1), np.uint32(1))
        mid_ks = _uk_to_ks(mid)

        @pl.when(active)
        def _():
            count_fn(mid_ks)

        cnt = _tree_sum(cntbuf[pl.ds(0, 16)])[0]
        geK = cnt >= K
        hitw = jnp.logical_and(active, jnp.logical_and(geK, cnt <= CAP))
        lo2 = jnp.where(jnp.logical_and(active, geK), mid, lo)
        hi2 = jnp.where(jnp.logical_and(active, jnp.logical_not(geK)),
                        mid - np.uint32(1), hi)
        conv = jnp.logical_and(jnp.logical_and(active,
                                               jnp.logical_not(hitw)),
                               lo2 == hi2)
        GTH2 = jnp.where(hitw, mid - np.uint32(1),
                         jnp.where(conv, lo2, GTH))
        EQLO2 = jnp.where(hitw, mid, jnp.where(conv, lo2, EQLO))
        capped2 = jnp.where(conv, 1, capped_i)
        done2 = jnp.where(jnp.logical_or(hitw, conv), 1, done_i)
        return (lo2, hi2, done2, GTH2, EQLO2, capped2)

    _, _, _, GTH, EQLO, capped_i = lax.fori_loop(
        0, 33, step, (lo0, hi0, jnp.int32(0), lo0, lo0, jnp.int32(0)))

    m1_ks = _uk_to_ks(GTH + np.uint32(1))

    @pl.when(capped_i == 1)
    def _():
        count_fn(m1_ks)

    m = jnp.where(capped_i == 1, _tree_sum(cntbuf[pl.ds(0, 16)])[0], 0)
    return GTH, EQLO, capped_i, m


def _compact_pair(dstK, dstI, gob, eob, u, idxv, gthv, eqlov):
    _append(dstK, dstI, gob, u, idxv, u > gthv)
    eq = jnp.logical_and(u >= eqlov, u <= gthv)
    ape = _sload(eob, 0) < K
    apev = _bc(jnp.where(ape, 1, 0), jnp.int32)
    eqm = (jnp.where(eq, 1, 0) & apev) == 1
    _append(dstK, dstI, eob, u, idxv, eqm)


def _bisect_select(src_ref, mode, bmax_ref, nblk, nvec, sortK_ref, sortI_ref,
                   cntbuf, gob, eob, fbK, fbI, fob):
    li = _li()
    zk = _bc(IMIN, jnp.int32)
    zi = jnp.zeros((16,), jnp.int32)

    def zbody(v, _):
        sortK_ref[pl.ds(16 * v, 16)] = zk
        sortI_ref[pl.ds(16 * v, 16)] = zi
        return 0
    lax.fori_loop(0, SORTN // 16, zbody, 0)

    accmin = _bc(np.int32(2**31 - 1), jnp.int32)
    accmax = _bc(IMIN, jnp.int32)
    for v in range((nblk + 15) // 16):
        w = bmax_ref[pl.ds(16 * v, 16)]
        valid = (li + 16 * v) < nblk
        accmin = jnp.minimum(accmin, jnp.where(valid, w, np.int32(2**31 - 1)))
        accmax = jnp.maximum(accmax, jnp.where(valid, w, IMIN))
    t0k = lax.rev(_tree_min(accmin), (0,))[0]
    gmaxu = _ks_to_u(lax.rev(_tree_max(accmax), (0,))[0])

    _count_ge(src_ref, mode, bmax_ref, nblk, t0k, cntbuf)
    cnt0 = _tree_sum(cntbuf[pl.ds(0, 16)])[0]
    fast = jnp.logical_and(cnt0 >= K, cnt0 <= FCAP)

    @pl.when(fast)
    def _():
        fob[pl.ds(0, 16)] = zi
        t0v = _bc(t0k, jnp.int32)

        def fblk(j, _):
            u = _ld_ks(src_ref, 16 * j, mode)
            _append(fbK, fbI, fob, u, li + 16 * j, u >= t0v, cap=FBUF)
            return 0
        lax.fori_loop(0, nvec, fblk, 0)
        flen = _sload(fob, 0)
        fbK[pl.ds(flen, 16)] = zk
        fbI[pl.ds(flen, 16)] = zi
        nvb = (flen + 15) >> 4

        def bcount(thr_ks):
            thrv = _bc(thr_ks, jnp.int32)

            def b(j, a):
                return a + jnp.where(fbK[pl.ds(16 * j, 16)] >= thrv, 1, 0)
            cntbuf[pl.ds(0, 16)] = lax.fori_loop(
                0, nvb, b, jnp.zeros((16,), jnp.int32))

        GTH, EQLO, capped_i, m = _bisect_core(bcount, cntbuf,
                                              _ks_to_u(t0k), gmaxu)
        gob[pl.ds(0, 16)] = zi
        eob[pl.ds(0, 16)] = _bc(m, jnp.int32)
        gthv = _bc(_uk_to_ks(GTH), jnp.int32)
        eqlov = _bc(_uk_to_ks(EQLO), jnp.int32)

        def cb(j, _):
            u = fbK[pl.ds(16 * j, 16)]
            iv = fbI[pl.ds(16 * j, 16)]
            _compact_pair(sortK_ref, sortI_ref, gob, eob, u, iv, gthv, eqlov)
            return 0
        lax.fori_loop(0, nvb, cb, 0)

    @pl.when(jnp.logical_not(fast))
    def _():
        def fcount(thr_ks):
            _count_ge(src_ref, mode, bmax_ref, nblk, thr_ks, cntbuf)

        GTH, EQLO, capped_i, m = _bisect_core(
            fcount, cntbuf, np.uint32(0x3F800000), gmaxu)
        gob[pl.ds(0, 16)] = zi
        eob[pl.ds(0, 16)] = _bc(m, jnp.int32)
        eqlo_ks = _uk_to_ks(EQLO)
        gthv = _bc(_uk_to_ks(GTH), jnp.int32)
        eqlov = _bc(eqlo_ks, jnp.int32)

        def cblk(bi, _):
            bm = _sload(bmax_ref, bi)

            @pl.when(bm >= eqlo_ks)
            def _():
                base = bi * BLK
                for jj in range(BLK):
                    u = _ld_ks(src_ref, 16 * (base + jj), mode)
                    idxv = li + 16 * (base + jj)
                    _compact_pair(sortK_ref, sortI_ref, gob, eob, u, idxv,
                                  gthv, eqlov)
            return 0
        lax.fori_loop(0, nblk, cblk, 0)


def _bitonic_sort_512(sortK_ref, sortI_ref):
    li = _li()

    def xlayer(kk, j, logkk, logj):
        jr = j // 16
        logjr = logj - 4

        def body(i, _):
            r1 = (i & (jr - 1)) | ((i >> logjr) << (logjr + 1))
            dscv = _bc((r1 >> (logkk - 4)) & 1, jnp.int32)
            o1 = 16 * r1
            o2 = o1 + j
            K1 = sortK_ref[pl.ds(o1, 16)]
            K2 = sortK_ref[pl.ds(o2, 16)]
            I1 = sortI_ref[pl.ds(o1, 16)]
            I2 = sortI_ref[pl.ds(o2, 16)]
            better1 = jnp.logical_or(K1 > K2,
                                     jnp.logical_and(K1 == K2, I1 < I2))
            keep = (jnp.where(better1, 1, 0) ^ dscv) == 1
            sortK_ref[pl.ds(o1, 16)] = jnp.where(keep, K1, K2)
            sortK_ref[pl.ds(o2, 16)] = jnp.where(keep, K2, K1)
            sortI_ref[pl.ds(o1, 16)] = jnp.where(keep, I1, I2)
            sortI_ref[pl.ds(o2, 16)] = jnp.where(keep, I2, I1)
            return 0
        lax.fori_loop(0, 16, body, 0)

    def llayer(kk, jl, logkk):
        pidx = li ^ jl
        upper = (li & jl) != 0

        upv = jnp.where(upper, 1, 0)

        def body(r, _):
            o = 16 * r
            Kv = sortK_ref[pl.ds(o, 16)]
            Iv = sortI_ref[pl.ds(o, 16)]
            pK = _laneperm(Kv, pidx)
            pI = _laneperm(Iv, pidx)
            if logkk < 4:
                flipv = ((li >> logkk) & 1) ^ upv
            else:
                flipv = _bc((r >> (logkk - 4)) & 1, jnp.int32) ^ upv
            better = jnp.logical_or(Kv > pK,
                                    jnp.logical_and(Kv == pK, Iv < pI))
            keep = (jnp.where(better, 1, 0) ^ flipv) == 1
            sortK_ref[pl.ds(o, 16)] = jnp.where(keep, Kv, pK)
            sortI_ref[pl.ds(o, 16)] = jnp.where(keep, Iv, pI)
            return 0
        lax.fori_loop(0, 32, body, 0)

    kk = 2
    while kk <= SORTN:
        logkk = kk.bit_length() - 1
        j = kk // 2
        while j >= 1:
            if j >= 16:
                xlayer(kk, j, logkk, j.bit_length() - 1)
            else:
                llayer(kk, j, logkk)
            j //= 2
        kk *= 2


def _extract_sorted(sortK_ref, sortI_ref, candI2_ref, tops_ref):
    for o in range(BG // 16):
        ii = sortI_ref[pl.ds(16 * o, 16)]
        candI2_ref[o // 8, pl.ds((o % 8) * 16, 16)] = ii
        if o < B // 16:
            ki = sortK_ref[pl.ds(16 * o, 16)]
            tops_ref[pl.ds(16 * o, 16)] = lax.bitcast_convert_type(
                ki, jnp.float32)


def _gather_boxes(planes, candI2_ref, dsts, sem):
    cps = []
    for g in range(3):
        for plane, dst in zip(planes, dsts):
            cp = pltpu.make_async_copy(plane.at[candI2_ref.at[g]],
                                       dst.at[pl.ds(128 * g, 128)], sem)
            cp.start()
            cps.append(cp)
    for cp in cps:
        cp.wait()


def _sc_main_body(scoresT, x1p, y1p, x2p, y2p,
                  outS, ob1, ob2, ob3, ob4,
                  scores_v, bmax, sortK, sortI, candI2, tops,
                  bx1, by1, bx2, by2, ar, ovlp, suppv, keepv, outbuf,
                  cntbuf, gob, eob, fbK, fbI, fob, sem):
    wid = lax.axis_index("s") * 2 + lax.axis_index("c")
    li = _li()
    zi = jnp.zeros((16,), jnp.int32)

    def class_body(t, _):
        c = wid + NW * t

        @pl.when(c < C)
        def _():
            pltpu.sync_copy(scoresT.at[pl.ds(c * N, N)], scores_v)
            _fill_ks(scores_v, None, bmax, NBLK, "f32mask")
            _bisect_select(scores_v, "f32mask", bmax, NBLK, NVEC, sortK,
                           sortI, cntbuf, gob, eob, fbK, fbI, fob)
            _bitonic_sort_512(sortK, sortI)
            _extract_sorted(sortK, sortI, candI2, tops)
            _gather_boxes((x1p, y1p, x2p, y2p), candI2,
                          (bx1, by1, bx2, by2), sem)

            def area_body(v, _):
                o = pl.ds(16 * v, 16)
                ar[o] = (bx2[o] - bx1[o]) * (by2[o] - by1[o])
                return 0
            lax.fori_loop(0, B // 16, area_body, 0)

            def row_body(i):
                w0 = i >> 4
                xi1 = _sload(bx1, i)
                yi1 = _sload(by1, i)
                xi2 = _sload(bx2, i)
                yi2 = _sload(by2, i)
                ai = _sload(ar, i)

                def col_body(v, _):
                    o = pl.ds(16 * v, 16)
                    X1 = bx1[o]
                    Y1 = by1[o]
                    X2 = bx2[o]
                    Y2 = by2[o]
                    A = ar[o]
                    iw = jnp.maximum(jnp.minimum(xi2, X2) - jnp.maximum(xi1, X1),
                                     0.0)
                    ih = jnp.maximum(jnp.minimum(yi2, Y2) - jnp.maximum(yi1, Y1),
                                     0.0)
                    inter = iw * ih
                    iou = inter / (ai + A - inter + 1e-8)
                    cm = jnp.logical_and(iou > 0.5, (li + 16 * v) > _bc(i, jnp.int32))
                    ovlp[pl.ds(i * B + 16 * v, 16)] = jnp.where(cm, 1, 0)
                    return 0
                lax.fori_loop(w0, B // 16, col_body, 0)

            plsc.parallel_loop(0, K, 1, unroll=2)(row_body)

            def zs_body(v, _):
                suppv[pl.ds(16 * v, 16)] = zi
                keepv[pl.ds(16 * v, 16)] = zi
                return 0
            lax.fori_loop(0, B // 16, zs_body, 0)

            def nms_body(i, _):
                w = i >> 4
                sup = _sload(suppv, i)
                live = jnp.logical_and(sup == 0, _sload(tops, i) > 0.0)
                kw = keepv[pl.ds(i, 16)]
                keepv[pl.ds(i, 16)] = jnp.where(
                    li == 0, _bc(jnp.where(live, 1, 0), jnp.int32), kw)

                @pl.when(live)
                def _():
                    def sb(v, _):
                        o = pl.ds(16 * v, 16)
                        suppv[o] = suppv[o] | ovlp[pl.ds(i * B + 16 * v, 16)]
                        return 0
                    lax.fori_loop(w, B // 16, sb, 0)
                return 0
            lax.fori_loop(0, K, nms_body, 0)

            for v in range(B // 16):
                kvec = keepv[pl.ds(16 * v, 16)]
                sv = tops[pl.ds(16 * v, 16)]
                colid = li + 16 * v
                outv = jnp.where(colid < K,
                                 jnp.where(kvec == 1, sv, jnp.float32(-1.0)),
                                 jnp.float32(-0.0))
                outbuf[pl.ds(16 * v, 16)] = outv

            pltpu.sync_copy(outbuf, outS.at[pl.ds(c * B, B)])
            pltpu.sync_copy(bx1.at[pl.ds(0, B)], ob1.at[pl.ds(c * B, B)])
            pltpu.sync_copy(by1.at[pl.ds(0, B)], ob2.at[pl.ds(c * B, B)])
            pltpu.sync_copy(bx2.at[pl.ds(0, B)], ob3.at[pl.ds(c * B, B)])
            pltpu.sync_copy(by2.at[pl.ds(0, B)], ob4.at[pl.ds(c * B, B)])
        return 0

    lax.fori_loop(0, 3, class_body, 0)


_sc_main = functools.partial(
    pl.kernel,
    out_type=[jax.ShapeDtypeStruct((C * B,), jnp.float32)] * 5,
    mesh=plsc.VectorSubcoreMesh(core_axis_name="c", subcore_axis_name="s"),
    scratch_types=[
        pltpu.VMEM((N,), jnp.float32),
        pltpu.VMEM((NBLK + 16,), jnp.int32),
        pltpu.VMEM((SORTN,), jnp.int32),
        pltpu.VMEM((SORTN,), jnp.int32),
        pltpu.VMEM((3, 128), jnp.int32),
        pltpu.VMEM((320,), jnp.float32),
        pltpu.VMEM((BG,), jnp.float32),
        pltpu.VMEM((BG,), jnp.float32),
        pltpu.VMEM((BG,), jnp.float32),
        pltpu.VMEM((BG,), jnp.float32),
        pltpu.VMEM((BG,), jnp.float32),
        pltpu.VMEM((K * B + 64,), jnp.int32),
        pltpu.VMEM((B + 64,), jnp.int32),
        pltpu.VMEM((B + 64,), jnp.int32),
        pltpu.VMEM((B,), jnp.float32),
        pltpu.VMEM((16,), jnp.int32),
        pltpu.VMEM((16,), jnp.int32),
        pltpu.VMEM((16,), jnp.int32),
        pltpu.VMEM((FBUF,), jnp.int32),
        pltpu.VMEM((FBUF,), jnp.int32),
        pltpu.VMEM((16,), jnp.int32),
        pltpu.SemaphoreType.DMA,
    ],
)(_sc_main_body)


def _sc_merge_body(flatS, fb1, fb2, fb3, fb4,
                   fscore, fidx, fo1, fo2, fo3, fo4,
                   scores_v, ksbuf, bmax, sortK, sortI, candI2, tops,
                   bx1, by1, bx2, by2, cntbuf, gob, eob, fbK, fbI, fob, sem):
    wid = lax.axis_index("s") * 2 + lax.axis_index("c")

    @pl.when(wid == 0)
    def _():
        pltpu.sync_copy(flatS, scores_v)
        _fill_ks(scores_v, ksbuf, bmax, NBLK_M, "f32raw")
        _bisect_select(ksbuf, "ks", bmax, NBLK_M, NVEC_M, sortK, sortI,
                       cntbuf, gob, eob, fbK, fbI, fob)
        _bitonic_sort_512(sortK, sortI)
        _extract_sorted(sortK, sortI, candI2, tops)
        _gather_boxes((fb1, fb2, fb3, fb4), candI2,
                      (bx1, by1, bx2, by2), sem)
        pltpu.sync_copy(tops.at[pl.ds(0, B)], fscore)
        pltpu.sync_copy(candI2.at[0], fidx.at[pl.ds(0, 128)])
        pltpu.sync_copy(candI2.at[1], fidx.at[pl.ds(128, 128)])
        pltpu.sync_copy(candI2.at[2], fidx.at[pl.ds(256, 128)])
        pltpu.sync_copy(bx1.at[pl.ds(0, B)], fo1)
        pltpu.sync_copy(by1.at[pl.ds(0, B)], fo2)
        pltpu.sync_copy(bx2.at[pl.ds(0, B)], fo3)
        pltpu.sync_copy(by2.at[pl.ds(0, B)], fo4)


_sc_merge = functools.partial(
    pl.kernel,
    out_type=[jax.ShapeDtypeStruct((B,), jnp.float32),
              jax.ShapeDtypeStruct((BG,), jnp.int32)] +
             [jax.ShapeDtypeStruct((B,), jnp.float32)] * 4,
    mesh=plsc.VectorSubcoreMesh(core_axis_name="c", subcore_axis_name="s"),
    scratch_types=[
        pltpu.VMEM((C * B,), jnp.float32),
        pltpu.VMEM((C * B,), jnp.int32),
        pltpu.VMEM((NBLK_M + 16,), jnp.int32),
        pltpu.VMEM((SORTN,), jnp.int32),
        pltpu.VMEM((SORTN,), jnp.int32),
        pltpu.VMEM((3, 128), jnp.int32),
        pltpu.VMEM((320,), jnp.float32),
        pltpu.VMEM((BG,), jnp.float32),
        pltpu.VMEM((BG,), jnp.float32),
        pltpu.VMEM((BG,), jnp.float32),
        pltpu.VMEM((BG,), jnp.float32),
        pltpu.VMEM((16,), jnp.int32),
        pltpu.VMEM((16,), jnp.int32),
        pltpu.VMEM((16,), jnp.int32),
        pltpu.VMEM((FBUF,), jnp.int32),
        pltpu.VMEM((FBUF,), jnp.int32),
        pltpu.VMEM((16,), jnp.int32),
        pltpu.SemaphoreType.DMA,
    ],
)(_sc_merge_body)


@jax.jit
def kernel(classifications, regressions, anchors):
    x1p, y1p, x2p, y2p = _decode_boxes(anchors[0], regressions[0])
    scoresT = jnp.transpose(classifications[0]).reshape(-1)
    outS, ob1, ob2, ob3, ob4 = _sc_main(scoresT, x1p, y1p, x2p, y2p)
    fs, fidx, f1, f2, f3, f4 = _sc_merge(outS, ob1, ob2, ob3, ob4)
    final_scores = fs[:K]
    final_labels = (fidx[:K] // B).astype(jnp.int32)
    final_boxes = jnp.stack([f1[:K], f2[:K], f3[:K], f4[:K]], axis=-1)
    return (final_scores, final_labels, final_boxes)
